# Initial kernel scaffold; baseline (speedup 1.0000x reference)
import jax
import jax.numpy as jnp
from jax import lax
from jax.experimental import pallas as pl
from jax.experimental.pallas import tpu as pltpu
from jax.experimental.pallas import tpu_sc as plsc

N = 50000
E = 800000
D_IN = 39
H = 128
ED = 2

BN = 2000
CH = 32
NCH = H // CH
B = 500
SUB = 125
NSUB = B // SUB
NBLK = E // B
NSC = 16
BLK_PER_TILE = NBLK // NSC
ROWS_PER_TILE = N // NSC


def _input_tables_call(xp, Wp, b_in2, Wcat, bcat2):
    grid = (N // BN,)
    outs = ([jax.ShapeDtypeStruct((N, H), jnp.float32)] +
            [jax.ShapeDtypeStruct((N, CH), jnp.float32) for _ in range(4 * NCH)])

    def body(x_ref, wp_ref, b_ref, wc_ref, bc_ref, h_ref, *tab_refs):
        hb = jnp.maximum(
            jnp.dot(x_ref[...], wp_ref[...], preferred_element_type=jnp.float32)
            + b_ref[...], 0.0)
        h_ref[...] = hb
        full = jnp.dot(hb, wc_ref[...], preferred_element_type=jnp.float32) + bc_ref[...]
        for t in range(4 * NCH):
            tab_refs[t][...] = full[:, t * CH:(t + 1) * CH]

    return pl.pallas_call(
        body,
        grid=grid,
        in_specs=[
            pl.BlockSpec((BN, H), lambda i: (i, 0)),
            pl.BlockSpec((H, H), lambda i: (0, 0)),
            pl.BlockSpec((1, H), lambda i: (0, 0)),
            pl.BlockSpec((H, 4 * NCH * CH), lambda i: (0, 0)),
            pl.BlockSpec((1, 4 * NCH * CH), lambda i: (0, 0)),
        ],
        out_specs=([pl.BlockSpec((BN, H), lambda i: (i, 0))] +
                   [pl.BlockSpec((BN, CH), lambda i: (i, 0))] * (4 * NCH)),
        out_shape=outs,
    )(xp, Wp, b_in2, Wcat, bcat2)


def _tables_call(h, Wcat, bcat2):
    grid = (N // BN,)
    outs = [jax.ShapeDtypeStruct((N, CH), jnp.float32) for _ in range(4 * NCH)]

    def body(h_ref, wc_ref, bc_ref, *tab_refs):
        full = jnp.dot(h_ref[...], wc_ref[...],
                       preferred_element_type=jnp.float32) + bc_ref[...]
        for t in range(4 * NCH):
            tab_refs[t][...] = full[:, t * CH:(t + 1) * CH]

    return pl.pallas_call(
        body,
        grid=grid,
        in_specs=[
            pl.BlockSpec((BN, H), lambda i: (i, 0)),
            pl.BlockSpec((H, 4 * NCH * CH), lambda i: (0, 0)),
            pl.BlockSpec((1, 4 * NCH * CH), lambda i: (0, 0)),
        ],
        out_specs=[pl.BlockSpec((BN, CH), lambda i: (i, 0))] * (4 * NCH),
        out_shape=outs,
    )(h, Wcat, bcat2)


def _combine_call(h, sa, se, dega2, dege2, aW2, ab2, eW2, eb2, g2, bl2, cW, cb2):
    grid = (N // BN,)

    def body(h_ref, sa_ref, se_ref, da_ref, de_ref, aw_ref, ab_ref,
             ew_ref, eb_ref, g_ref, bl_ref, cw_ref, cb_ref, o_ref):
        sa_full = jnp.concatenate([sa_ref[c] for c in range(NCH)], axis=-1)
        se_full = jnp.concatenate([se_ref[c] for c in range(NCH)], axis=-1)
        h1 = (h_ref[...]
              + jnp.dot(sa_full, aw_ref[...], preferred_element_type=jnp.float32)
              + da_ref[0][:, None] * ab_ref[...]
              + jnp.dot(se_full, ew_ref[...], preferred_element_type=jnp.float32)
              + de_ref[0][:, None] * eb_ref[...])
        mu = jnp.mean(h1, axis=-1, keepdims=True)
        var = jnp.mean((h1 - mu) ** 2, axis=-1, keepdims=True)
        t = (h1 - mu) / jnp.sqrt(var + 1e-5) * g_ref[...] + bl_ref[...]
        o_ref[...] = t + jnp.maximum(
            jnp.dot(t, cw_ref[...], preferred_element_type=jnp.float32)
            + cb_ref[...], 0.0)

    return pl.pallas_call(
        body,
        grid=grid,
        in_specs=[
            pl.BlockSpec((BN, H), lambda i: (i, 0)),
            pl.BlockSpec((NCH, BN, CH), lambda i: (0, i, 0)),
            pl.BlockSpec((NCH, BN, CH), lambda i: (0, i, 0)),
            pl.BlockSpec((1, BN), lambda i: (i, 0)),
            pl.BlockSpec((1, BN), lambda i: (i, 0)),
            pl.BlockSpec((H, H), lambda i: (0, 0)),
            pl.BlockSpec((1, H), lambda i: (0, 0)),
            pl.BlockSpec((H, H), lambda i: (0, 0)),
            pl.BlockSpec((1, H), lambda i: (0, 0)),
            pl.BlockSpec((1, H), lambda i: (0, 0)),
            pl.BlockSpec((1, H), lambda i: (0, 0)),
            pl.BlockSpec((H, H), lambda i: (0, 0)),
            pl.BlockSpec((1, H), lambda i: (0, 0)),
        ],
        out_specs=pl.BlockSpec((BN, H), lambda i: (i, 0)),
        out_shape=jax.ShapeDtypeStruct((N, H), jnp.float32),
    )(h, sa, se, dega2, dege2, aW2, ab2, eW2, eb2, g2, bl2, cW, cb2)


def _make_conv(want_deg):
    mesh = plsc.VectorSubcoreMesh(core_axis_name="c", subcore_axis_name="s")
    out_type = [jax.ShapeDtypeStruct((NCH, N, CH), jnp.float32)]
    if want_deg:
        out_type.append(jax.ShapeDtypeStruct((N,), jnp.float32))
    scratch = [
        pltpu.VMEM_SHARED((N, CH), jnp.float32),
        pltpu.VMEM_SHARED((N,), jnp.float32),
        pltpu.VMEM((NSUB, SUB), jnp.int32),
        pltpu.VMEM((NSUB, SUB), jnp.int32),
        pltpu.VMEM((B,), jnp.float32),
        pltpu.VMEM((B,), jnp.float32),
        pltpu.SMEM((B,), jnp.float32),
        pltpu.SMEM((B,), jnp.float32),
        pltpu.VMEM((2, CH), jnp.float32),
        pltpu.VMEM((B, CH), jnp.float32),
        pltpu.VMEM((B, CH), jnp.float32),
        pltpu.VMEM((B, CH), jnp.float32),
        pltpu.VMEM((SUB,), jnp.float32),
        pltpu.SemaphoreType.DMA,
        pltpu.SemaphoreType.DMA,
    ]

    def body(p0, p1, p2, p3, q0, q1, q2, q3, dst3, src3, ea0r, ea1r,
             wattr, z2d, z1d, ones_h, s_out, *rest):
        if want_deg:
            deg_out = rest[0]
            rest = rest[1:]
        (acc, deg_acc, dst_b, src_b, ea0_v, ea1_v, ea0_s, ea1_s,
         w_v, pd, qs, tb, ones_v, sem1, sem2) = rest
        cid = lax.axis_index("c")
        sid = lax.axis_index("s")
        row0 = sid * ROWS_PER_TILE
        pts = (p0, p1, p2, p3)
        qts = (q0, q1, q2, q3)
        pltpu.sync_copy(ones_h, ones_v)

        def do_pass(chunk, with_deg):
            pt = pts[chunk]
            qt = qts[chunk]
            pltpu.sync_copy(z2d, acc.at[pl.ds(row0, ROWS_PER_TILE)])
            if with_deg:
                @pl.when(sid == 0)
                def _():
                    pltpu.sync_copy(z1d, deg_acc)
            pltpu.sync_copy(wattr.at[chunk], w_v)
            plsc.subcore_barrier()
            w00 = w_v[0, pl.ds(0, 16)]
            w01 = w_v[0, pl.ds(16, 16)]
            w10 = w_v[1, pl.ds(0, 16)]
            w11 = w_v[1, pl.ds(16, 16)]
            blk0 = sid * BLK_PER_TILE

            @pl.loop(0, BLK_PER_TILE)
            def _blk(jj):
                j = blk0 + jj
                pltpu.sync_copy(dst3.at[j], dst_b)
                pltpu.sync_copy(src3.at[j], src_b)
                pltpu.sync_copy(ea0r.at[j], ea0_v)
                pltpu.sync_copy(ea1r.at[j], ea1_v)
                pltpu.sync_copy(ea0_v, ea0_s)
                pltpu.sync_copy(ea1_v, ea1_s)
                gd = [pltpu.async_copy(pt.at[dst_b.at[k]],
                                       pd.at[pl.ds(k * SUB, SUB)], sem1)
                      for k in range(NSUB)]
                gs = [pltpu.async_copy(qt.at[src_b.at[k]],
                                       qs.at[pl.ds(k * SUB, SUB)], sem2)
                      for k in range(NSUB)]
                for d in gd:
                    d.wait()
                for d in gs:
                    d.wait()

                @pl.loop(0, B)
                def _edge(e):
                    a0 = ea0_s[e]
                    a1 = ea1_s[e]
                    p_lo = pd[e, pl.ds(0, 16)]
                    p_hi = pd[e, pl.ds(16, 16)]
                    q_lo = qs[e, pl.ds(0, 16)]
                    q_hi = qs[e, pl.ds(16, 16)]
                    tb[e, pl.ds(0, 16)] = jnp.maximum(
                        p_lo + q_lo + a0 * w00 + a1 * w10, 0.0)
                    tb[e, pl.ds(16, 16)] = jnp.maximum(
                        p_hi + q_hi + a0 * w01 + a1 * w11, 0.0)

                for k in range(NSUB):
                    pltpu.sync_copy(tb.at[pl.ds(k * SUB, SUB)],
                                    acc.at[dst_b.at[k]], add=True)
                if with_deg:
                    for k in range(NSUB):
                        pltpu.sync_copy(ones_v, deg_acc.at[dst_b.at[k]],
                                        add=True)

            plsc.subcore_barrier()
            pltpu.sync_copy(acc.at[pl.ds(row0, ROWS_PER_TILE)],
                            s_out.at[chunk].at[pl.ds(row0, ROWS_PER_TILE)])
            if with_deg:
                @pl.when(sid == 0)
                def _():
                    pltpu.sync_copy(deg_acc, deg_out)

        @pl.when(cid == 0)
        def _():
            do_pass(0, want_deg)
            do_pass(1, False)

        @pl.when(cid == 1)
        def _():
            do_pass(2, False)
            do_pass(3, False)

    return pl.kernel(body, out_type=out_type, mesh=mesh,
                     scratch_types=scratch)


_conv_cache = {}


def _get_conv(want_deg):
    if want_deg not in _conv_cache:
        _conv_cache[want_deg] = _make_conv(want_deg)
    return _conv_cache[want_deg]


def kernel(x, ally_edge_index, ally_edge_attr, enc_edge_index, enc_edge_attr,
           W_in, b_in, ally_W1_0, ally_b1_0, ally_W2_0, ally_b2_0,
           enc_W1_0, enc_b1_0, enc_W2_0, enc_b2_0, ln_g_0, ln_b_0,
           comb_W_0, comb_b_0, ally_W1_1, ally_b1_1, ally_W2_1, ally_b2_1,
           enc_W1_1, enc_b1_1, enc_W2_1, enc_b2_1, ln_g_1, ln_b_1,
           comb_W_1, comb_b_1):
    f32 = jnp.float32
    conv_deg = _get_conv(True)
    conv_nodeg = _get_conv(False)

    xp = jnp.pad(x, ((0, 0), (0, H - D_IN)))
    Wp = jnp.pad(W_in, ((0, H - D_IN), (0, 0)))

    def edge_prep(ei, ea):
        dst3 = ei[1].reshape(NBLK, NSUB, SUB)
        src3 = ei[0].reshape(NBLK, NSUB, SUB)
        ea0r = ea[:, 0].reshape(NBLK, B)
        ea1r = ea[:, 1].reshape(NBLK, B)
        return dst3, src3, ea0r, ea1r

    a_dst3, a_src3, a_ea0, a_ea1 = edge_prep(ally_edge_index, ally_edge_attr)
    e_dst3, e_src3, e_ea0, e_ea1 = edge_prep(enc_edge_index, enc_edge_attr)

    def wcat_prep(aW1, ab1, eW1, eb1):
        Wcat = jnp.concatenate(
            [aW1[:H], aW1[H:2 * H], eW1[:H], eW1[H:2 * H]], axis=1)
        bcat = jnp.concatenate(
            [ab1, jnp.zeros((H,), f32), eb1, jnp.zeros((H,), f32)])
        wattr_a = aW1[2 * H:].reshape(ED, NCH, CH).transpose(1, 0, 2)
        wattr_e = eW1[2 * H:].reshape(ED, NCH, CH).transpose(1, 0, 2)
        return Wcat, bcat.reshape(1, -1), wattr_a, wattr_e

    Wcat0, bcat0, wattr_a0, wattr_e0 = wcat_prep(ally_W1_0, ally_b1_0,
                                                 enc_W1_0, enc_b1_0)
    Wcat1, bcat1, wattr_a1, wattr_e1 = wcat_prep(ally_W1_1, ally_b1_1,
                                                 enc_W1_1, enc_b1_1)

    z2d = jnp.zeros((ROWS_PER_TILE, CH), f32)
    z1d = jnp.zeros((N,), f32)
    ones_h = jnp.ones((SUB,), f32)

    def r2(v):
        return v.reshape(1, -1)

    outs = _input_tables_call(xp, Wp, r2(b_in), Wcat0, bcat0)
    h = outs[0]
    tabs = outs[1:]
    ap, aq = tabs[0:NCH], tabs[NCH:2 * NCH]
    ep, eq = tabs[2 * NCH:3 * NCH], tabs[3 * NCH:4 * NCH]

    sa, dega = conv_deg(*ap, *aq, a_dst3, a_src3, a_ea0, a_ea1,
                        wattr_a0, z2d, z1d, ones_h)
    se, dege = conv_deg(*ep, *eq, e_dst3, e_src3, e_ea0, e_ea1,
                        wattr_e0, z2d, z1d, ones_h)

    dega2 = dega.reshape(N // BN, BN)
    dege2 = dege.reshape(N // BN, BN)

    h = _combine_call(h, sa, se, dega2, dege2,
                      ally_W2_0, r2(ally_b2_0), enc_W2_0, r2(enc_b2_0),
                      r2(ln_g_0), r2(ln_b_0), comb_W_0, r2(comb_b_0))

    tabs = _tables_call(h, Wcat1, bcat1)
    ap, aq = tabs[0:NCH], tabs[NCH:2 * NCH]
    ep, eq = tabs[2 * NCH:3 * NCH], tabs[3 * NCH:4 * NCH]

    sa = conv_nodeg(*ap, *aq, a_dst3, a_src3, a_ea0, a_ea1,
                    wattr_a1, z2d, z1d, ones_h)
    se = conv_nodeg(*ep, *eq, e_dst3, e_src3, e_ea0, e_ea1,
                    wattr_e1, z2d, z1d, ones_h)

    h = _combine_call(h, sa, se, dega2, dege2,
                      ally_W2_1, r2(ally_b2_1), enc_W2_1, r2(enc_b2_1),
                      r2(ln_g_1), r2(ln_b_1), comb_W_1, r2(comb_b_1))
    return h


# speedup vs baseline: 1.9914x; 1.9914x over previous
import jax
import jax.numpy as jnp
from jax import lax
from jax.experimental import pallas as pl
from jax.experimental.pallas import tpu as pltpu
from jax.experimental.pallas import tpu_sc as plsc

N = 50000
E = 800000
D_IN = 39
H = 128
ED = 2

BN = 2000
CH = 16
NCH = H // CH
PPC = NCH // 2
B = 500
SUB = 125
NSUB = B // SUB
NBLK = E // B
NSC = 16
BLK_PER_TILE = NBLK // NSC
RPT = 3128
RPT_LAST = N - 15 * RPT


def _input_tables_call(xp, Wp, b_in2, Wcat, bcat2):
    BT = 1000
    grid = (N // BT,)
    outs = ([jax.ShapeDtypeStruct((N, H), jnp.float32)] +
            [jax.ShapeDtypeStruct((N, CH), jnp.float32) for _ in range(4 * NCH)])

    def body(x_ref, wp_ref, b_ref, wc_ref, bc_ref, h_ref, *tab_refs):
        hb = jnp.maximum(
            jnp.dot(x_ref[...], wp_ref[...], preferred_element_type=jnp.float32)
            + b_ref[...], 0.0)
        h_ref[...] = hb
        full = jnp.dot(hb, wc_ref[...], preferred_element_type=jnp.float32) + bc_ref[...]
        for t in range(4 * NCH):
            tab_refs[t][...] = full[:, t * CH:(t + 1) * CH]

    return pl.pallas_call(
        body,
        grid=grid,
        in_specs=[
            pl.BlockSpec((BT, H), lambda i: (i, 0)),
            pl.BlockSpec((H, H), lambda i: (0, 0)),
            pl.BlockSpec((1, H), lambda i: (0, 0)),
            pl.BlockSpec((H, 4 * NCH * CH), lambda i: (0, 0)),
            pl.BlockSpec((1, 4 * NCH * CH), lambda i: (0, 0)),
        ],
        out_specs=([pl.BlockSpec((BT, H), lambda i: (i, 0))] +
                   [pl.BlockSpec((BT, CH), lambda i: (i, 0))] * (4 * NCH)),
        out_shape=outs,
    )(xp, Wp, b_in2, Wcat, bcat2)


def _tables_call(h, Wcat, bcat2):
    BT = 1000
    grid = (N // BT,)
    outs = [jax.ShapeDtypeStruct((N, CH), jnp.float32) for _ in range(4 * NCH)]

    def body(h_ref, wc_ref, bc_ref, *tab_refs):
        full = jnp.dot(h_ref[...], wc_ref[...],
                       preferred_element_type=jnp.float32) + bc_ref[...]
        for t in range(4 * NCH):
            tab_refs[t][...] = full[:, t * CH:(t + 1) * CH]

    return pl.pallas_call(
        body,
        grid=grid,
        in_specs=[
            pl.BlockSpec((BT, H), lambda i: (i, 0)),
            pl.BlockSpec((H, 4 * NCH * CH), lambda i: (0, 0)),
            pl.BlockSpec((1, 4 * NCH * CH), lambda i: (0, 0)),
        ],
        out_specs=[pl.BlockSpec((BT, CH), lambda i: (i, 0))] * (4 * NCH),
        out_shape=outs,
    )(h, Wcat, bcat2)


def _combine_call(h, sa, se, dega2, dege2, aW2, ab2, eW2, eb2, g2, bl2, cW, cb2):
    grid = (N // BN,)

    def body(h_ref, sa_ref, se_ref, da_ref, de_ref, aw_ref, ab_ref,
             ew_ref, eb_ref, g_ref, bl_ref, cw_ref, cb_ref, o_ref):
        sa_full = jnp.concatenate([sa_ref[c] for c in range(NCH)], axis=-1)
        se_full = jnp.concatenate([se_ref[c] for c in range(NCH)], axis=-1)
        h1 = (h_ref[...]
              + jnp.dot(sa_full, aw_ref[...], preferred_element_type=jnp.float32)
              + da_ref[0, 0][:, None] * ab_ref[...]
              + jnp.dot(se_full, ew_ref[...], preferred_element_type=jnp.float32)
              + de_ref[0, 0][:, None] * eb_ref[...])
        mu = jnp.mean(h1, axis=-1, keepdims=True)
        var = jnp.mean((h1 - mu) ** 2, axis=-1, keepdims=True)
        t = (h1 - mu) / jnp.sqrt(var + 1e-5) * g_ref[...] + bl_ref[...]
        o_ref[...] = t + jnp.maximum(
            jnp.dot(t, cw_ref[...], preferred_element_type=jnp.float32)
            + cb_ref[...], 0.0)

    return pl.pallas_call(
        body,
        grid=grid,
        in_specs=[
            pl.BlockSpec((BN, H), lambda i: (i, 0)),
            pl.BlockSpec((NCH, BN, CH), lambda i: (0, i, 0)),
            pl.BlockSpec((NCH, BN, CH), lambda i: (0, i, 0)),
            pl.BlockSpec((1, 1, BN), lambda i: (i, 0, 0)),
            pl.BlockSpec((1, 1, BN), lambda i: (i, 0, 0)),
            pl.BlockSpec((H, H), lambda i: (0, 0)),
            pl.BlockSpec((1, H), lambda i: (0, 0)),
            pl.BlockSpec((H, H), lambda i: (0, 0)),
            pl.BlockSpec((1, H), lambda i: (0, 0)),
            pl.BlockSpec((1, H), lambda i: (0, 0)),
            pl.BlockSpec((1, H), lambda i: (0, 0)),
            pl.BlockSpec((H, H), lambda i: (0, 0)),
            pl.BlockSpec((1, H), lambda i: (0, 0)),
        ],
        out_specs=pl.BlockSpec((BN, H), lambda i: (i, 0)),
        out_shape=jax.ShapeDtypeStruct((N, H), jnp.float32),
    )(h, sa, se, dega2, dege2, aW2, ab2, eW2, eb2, g2, bl2, cW, cb2)


def _make_conv(want_deg):
    mesh = plsc.VectorSubcoreMesh(core_axis_name="c", subcore_axis_name="s",
                                  num_cores=2, num_subcores=NSC)
    out_type = [jax.ShapeDtypeStruct((NCH, N, CH), jnp.float32)]
    if want_deg:
        out_type.append(jax.ShapeDtypeStruct((N,), jnp.float32))
    scratch = [
        pltpu.VMEM_SHARED((N, CH), jnp.float32),
        pltpu.VMEM_SHARED((N,), jnp.float32),
        pltpu.VMEM((NSUB, SUB), jnp.int32),
        pltpu.VMEM((NSUB, SUB), jnp.int32),
        pltpu.VMEM((1, B), jnp.float32),
        pltpu.VMEM((1, B), jnp.float32),
        pltpu.VMEM((ED, CH), jnp.float32),
        pltpu.VMEM((B, CH), jnp.float32),
        pltpu.VMEM((B, CH), jnp.float32),
        pltpu.VMEM((B, CH), jnp.float32),
        pltpu.VMEM((SUB,), jnp.float32),
        pltpu.SemaphoreType.DMA,
        pltpu.SemaphoreType.DMA,
    ]

    def body(*args):
        pts = args[0:NCH]
        qts = args[NCH:2 * NCH]
        (dst3, src3, ea0r, ea1r, wattr, z2d, z1d, ones_h, s_out) = \
            args[2 * NCH:2 * NCH + 9]
        rest = args[2 * NCH + 9:]
        if want_deg:
            deg_out = rest[0]
            rest = rest[1:]
        (acc, deg_acc, dst_b, src_b, ea0_v, ea1_v,
         w_v, pd, qs, tb, ones_v, sem1, sem2) = rest
        cid = lax.axis_index("c")
        sid = lax.axis_index("s")
        row0 = sid * RPT
        row0_l = (NSC - 1) * RPT
        pltpu.sync_copy(ones_h, ones_v)

        def do_pass(chunk, with_deg):
            pt = pts[chunk]
            qt = qts[chunk]

            @pl.when(sid < NSC - 1)
            def _():
                pltpu.sync_copy(z2d, acc.at[pl.ds(row0, RPT)])

            @pl.when(sid == NSC - 1)
            def _():
                pltpu.sync_copy(z2d.at[pl.ds(0, RPT_LAST)],
                                acc.at[pl.ds(row0, RPT_LAST)])

            if with_deg:
                @pl.when(sid == 0)
                def _():
                    pltpu.sync_copy(z1d, deg_acc)
            pltpu.sync_copy(wattr.at[chunk], w_v)
            plsc.subcore_barrier()
            w0 = w_v[0, pl.ds(0, CH)]
            w1 = w_v[1, pl.ds(0, CH)]
            blk0 = sid * BLK_PER_TILE

            @pl.loop(0, BLK_PER_TILE)
            def _blk(jj):
                j = blk0 + jj
                pltpu.sync_copy(dst3.at[j], dst_b)
                pltpu.sync_copy(src3.at[j], src_b)
                pltpu.sync_copy(ea0r.at[j], ea0_v)
                pltpu.sync_copy(ea1r.at[j], ea1_v)
                gd = [pltpu.async_copy(pt.at[dst_b.at[k]],
                                       pd.at[pl.ds(k * SUB, SUB)], sem1)
                      for k in range(NSUB)]
                gs = [pltpu.async_copy(qt.at[src_b.at[k]],
                                       qs.at[pl.ds(k * SUB, SUB)], sem2)
                      for k in range(NSUB)]
                for d in gd:
                    d.wait()
                for d in gs:
                    d.wait()

                @pl.loop(0, (B + 15) // 16)
                def _grp(g):
                    e0 = jnp.minimum(g * 16, B - 16)
                    a0v = ea0_v[0, pl.ds(e0, 16)]
                    a1v = ea1_v[0, pl.ds(e0, 16)]
                    for i in range(16):
                        e = e0 + i
                        a0 = a0v[i]
                        a1 = a1v[i]
                        pv = pd[e, pl.ds(0, CH)]
                        qv = qs[e, pl.ds(0, CH)]
                        tb[e, pl.ds(0, CH)] = jnp.maximum(
                            pv + qv + a0 * w0 + a1 * w1, 0.0)

                for k in range(NSUB):
                    pltpu.sync_copy(tb.at[pl.ds(k * SUB, SUB)],
                                    acc.at[dst_b.at[k]], add=True)
                if with_deg:
                    for k in range(NSUB):
                        pltpu.sync_copy(ones_v, deg_acc.at[dst_b.at[k]],
                                        add=True)

            plsc.subcore_barrier()

            @pl.when(sid < NSC - 1)
            def _():
                pltpu.sync_copy(acc.at[pl.ds(row0, RPT)],
                                s_out.at[chunk].at[pl.ds(row0, RPT)])

            @pl.when(sid == NSC - 1)
            def _():
                pltpu.sync_copy(acc.at[pl.ds(row0, RPT_LAST)],
                                s_out.at[chunk].at[pl.ds(row0, RPT_LAST)])
            if with_deg:
                @pl.when(sid == 0)
                def _():
                    pltpu.sync_copy(deg_acc, deg_out)

        @pl.when(cid == 0)
        def _():
            for p in range(PPC):
                do_pass(p, want_deg and p == 0)

        @pl.when(cid == 1)
        def _():
            for p in range(PPC, NCH):
                do_pass(p, False)

    return pl.kernel(body, out_type=out_type, mesh=mesh,
                     scratch_types=scratch,
                     compiler_params=pltpu.CompilerParams(
                         use_tc_tiling_on_sc=False))


_conv_cache = {}


def _get_conv(want_deg):
    if want_deg not in _conv_cache:
        _conv_cache[want_deg] = _make_conv(want_deg)
    return _conv_cache[want_deg]


def kernel(x, ally_edge_index, ally_edge_attr, enc_edge_index, enc_edge_attr,
           W_in, b_in, ally_W1_0, ally_b1_0, ally_W2_0, ally_b2_0,
           enc_W1_0, enc_b1_0, enc_W2_0, enc_b2_0, ln_g_0, ln_b_0,
           comb_W_0, comb_b_0, ally_W1_1, ally_b1_1, ally_W2_1, ally_b2_1,
           enc_W1_1, enc_b1_1, enc_W2_1, enc_b2_1, ln_g_1, ln_b_1,
           comb_W_1, comb_b_1):
    f32 = jnp.float32
    conv_deg = _get_conv(True)
    conv_nodeg = _get_conv(False)

    xp = jnp.pad(x, ((0, 0), (0, H - D_IN)))
    Wp = jnp.pad(W_in, ((0, H - D_IN), (0, 0)))

    def edge_prep(ei, ea):
        dst3 = ei[1].reshape(NBLK, NSUB, SUB)
        src3 = ei[0].reshape(NBLK, NSUB, SUB)
        ea0r = ea[:, 0].reshape(NBLK, 1, B)
        ea1r = ea[:, 1].reshape(NBLK, 1, B)
        return dst3, src3, ea0r, ea1r

    a_dst3, a_src3, a_ea0, a_ea1 = edge_prep(ally_edge_index, ally_edge_attr)
    e_dst3, e_src3, e_ea0, e_ea1 = edge_prep(enc_edge_index, enc_edge_attr)

    def wcat_prep(aW1, ab1, eW1, eb1):
        Wcat = jnp.concatenate(
            [aW1[:H], aW1[H:2 * H], eW1[:H], eW1[H:2 * H]], axis=1)
        bcat = jnp.concatenate(
            [ab1, jnp.zeros((H,), f32), eb1, jnp.zeros((H,), f32)])
        wattr_a = aW1[2 * H:].reshape(ED, NCH, CH).transpose(1, 0, 2)
        wattr_e = eW1[2 * H:].reshape(ED, NCH, CH).transpose(1, 0, 2)
        return Wcat, bcat.reshape(1, -1), wattr_a, wattr_e

    Wcat0, bcat0, wattr_a0, wattr_e0 = wcat_prep(ally_W1_0, ally_b1_0,
                                                 enc_W1_0, enc_b1_0)
    Wcat1, bcat1, wattr_a1, wattr_e1 = wcat_prep(ally_W1_1, ally_b1_1,
                                                 enc_W1_1, enc_b1_1)

    z2d = jnp.zeros((RPT, CH), f32)
    z1d = jnp.zeros((N,), f32)
    ones_h = jnp.ones((SUB,), f32)

    def r2(v):
        return v.reshape(1, -1)

    outs = _input_tables_call(xp, Wp, r2(b_in), Wcat0, bcat0)
    h = outs[0]
    tabs = outs[1:]
    ap, aq = tabs[0:NCH], tabs[NCH:2 * NCH]
    ep, eq = tabs[2 * NCH:3 * NCH], tabs[3 * NCH:4 * NCH]

    sa, dega = conv_deg(*ap, *aq, a_dst3, a_src3, a_ea0, a_ea1,
                        wattr_a0, z2d, z1d, ones_h)
    se, dege = conv_deg(*ep, *eq, e_dst3, e_src3, e_ea0, e_ea1,
                        wattr_e0, z2d, z1d, ones_h)

    dega2 = dega.reshape(N // BN, 1, BN)
    dege2 = dege.reshape(N // BN, 1, BN)

    h = _combine_call(h, sa, se, dega2, dege2,
                      ally_W2_0, r2(ally_b2_0), enc_W2_0, r2(enc_b2_0),
                      r2(ln_g_0), r2(ln_b_0), comb_W_0, r2(comb_b_0))

    tabs = _tables_call(h, Wcat1, bcat1)
    ap, aq = tabs[0:NCH], tabs[NCH:2 * NCH]
    ep, eq = tabs[2 * NCH:3 * NCH], tabs[3 * NCH:4 * NCH]

    (sa,) = conv_nodeg(*ap, *aq, a_dst3, a_src3, a_ea0, a_ea1,
                       wattr_a1, z2d, z1d, ones_h)
    (se,) = conv_nodeg(*ep, *eq, e_dst3, e_src3, e_ea0, e_ea1,
                       wattr_e1, z2d, z1d, ones_h)

    h = _combine_call(h, sa, se, dega2, dege2,
                      ally_W2_1, r2(ally_b2_1), enc_W2_1, r2(enc_b2_1),
                      r2(ln_g_1), r2(ln_b_1), comb_W_1, r2(comb_b_1))
    return h


# speedup vs baseline: 2.1826x; 1.0960x over previous
import jax
import jax.numpy as jnp
from jax import lax
from jax.experimental import pallas as pl
from jax.experimental.pallas import tpu as pltpu
from jax.experimental.pallas import tpu_sc as plsc

N = 50000
E = 800000
D_IN = 39
H = 128
ED = 2

BN = 2000
CH = 16
NCH = H // CH
PPC = NCH // 2
B = 500
SUB = 125
NSUB = B // SUB
NBLK = E // B
NSC = 16
BLK_PER_TILE = NBLK // NSC
RPT = 3128
RPT_LAST = N - 15 * RPT


def _input_tables_call(xp, Wp, b_in2, Wcat, bcat2):
    BT = 1000
    grid = (N // BT,)
    outs = ([jax.ShapeDtypeStruct((N, H), jnp.float32)] +
            [jax.ShapeDtypeStruct((N, CH), jnp.float32) for _ in range(4 * NCH)])

    def body(x_ref, wp_ref, b_ref, wc_ref, bc_ref, h_ref, *tab_refs):
        hb = jnp.maximum(
            jnp.dot(x_ref[...], wp_ref[...], preferred_element_type=jnp.float32)
            + b_ref[...], 0.0)
        h_ref[...] = hb
        full = jnp.dot(hb, wc_ref[...], preferred_element_type=jnp.float32) + bc_ref[...]
        for t in range(4 * NCH):
            tab_refs[t][...] = full[:, t * CH:(t + 1) * CH]

    return pl.pallas_call(
        body,
        grid=grid,
        in_specs=[
            pl.BlockSpec((BT, H), lambda i: (i, 0)),
            pl.BlockSpec((H, H), lambda i: (0, 0)),
            pl.BlockSpec((1, H), lambda i: (0, 0)),
            pl.BlockSpec((H, 4 * NCH * CH), lambda i: (0, 0)),
            pl.BlockSpec((1, 4 * NCH * CH), lambda i: (0, 0)),
        ],
        out_specs=([pl.BlockSpec((BT, H), lambda i: (i, 0))] +
                   [pl.BlockSpec((BT, CH), lambda i: (i, 0))] * (4 * NCH)),
        out_shape=outs,
    )(xp, Wp, b_in2, Wcat, bcat2)


def _tables_call(h, Wcat, bcat2):
    BT = 1000
    grid = (N // BT,)
    outs = [jax.ShapeDtypeStruct((N, CH), jnp.float32) for _ in range(4 * NCH)]

    def body(h_ref, wc_ref, bc_ref, *tab_refs):
        full = jnp.dot(h_ref[...], wc_ref[...],
                       preferred_element_type=jnp.float32) + bc_ref[...]
        for t in range(4 * NCH):
            tab_refs[t][...] = full[:, t * CH:(t + 1) * CH]

    return pl.pallas_call(
        body,
        grid=grid,
        in_specs=[
            pl.BlockSpec((BT, H), lambda i: (i, 0)),
            pl.BlockSpec((H, 4 * NCH * CH), lambda i: (0, 0)),
            pl.BlockSpec((1, 4 * NCH * CH), lambda i: (0, 0)),
        ],
        out_specs=[pl.BlockSpec((BT, CH), lambda i: (i, 0))] * (4 * NCH),
        out_shape=outs,
    )(h, Wcat, bcat2)


def _combine_call(h, sa, se, dega2, dege2, aW2, ab2, eW2, eb2, g2, bl2, cW, cb2):
    grid = (N // BN,)

    def body(h_ref, sa_ref, se_ref, da_ref, de_ref, aw_ref, ab_ref,
             ew_ref, eb_ref, g_ref, bl_ref, cw_ref, cb_ref, o_ref):
        sa_full = jnp.concatenate([sa_ref[c] for c in range(NCH)], axis=-1)
        se_full = jnp.concatenate([se_ref[c] for c in range(NCH)], axis=-1)
        h1 = (h_ref[...]
              + jnp.dot(sa_full, aw_ref[...], preferred_element_type=jnp.float32)
              + da_ref[0, 0][:, None] * ab_ref[...]
              + jnp.dot(se_full, ew_ref[...], preferred_element_type=jnp.float32)
              + de_ref[0, 0][:, None] * eb_ref[...])
        mu = jnp.mean(h1, axis=-1, keepdims=True)
        var = jnp.mean((h1 - mu) ** 2, axis=-1, keepdims=True)
        t = (h1 - mu) / jnp.sqrt(var + 1e-5) * g_ref[...] + bl_ref[...]
        o_ref[...] = t + jnp.maximum(
            jnp.dot(t, cw_ref[...], preferred_element_type=jnp.float32)
            + cb_ref[...], 0.0)

    return pl.pallas_call(
        body,
        grid=grid,
        in_specs=[
            pl.BlockSpec((BN, H), lambda i: (i, 0)),
            pl.BlockSpec((NCH, BN, CH), lambda i: (0, i, 0)),
            pl.BlockSpec((NCH, BN, CH), lambda i: (0, i, 0)),
            pl.BlockSpec((1, 1, BN), lambda i: (i, 0, 0)),
            pl.BlockSpec((1, 1, BN), lambda i: (i, 0, 0)),
            pl.BlockSpec((H, H), lambda i: (0, 0)),
            pl.BlockSpec((1, H), lambda i: (0, 0)),
            pl.BlockSpec((H, H), lambda i: (0, 0)),
            pl.BlockSpec((1, H), lambda i: (0, 0)),
            pl.BlockSpec((1, H), lambda i: (0, 0)),
            pl.BlockSpec((1, H), lambda i: (0, 0)),
            pl.BlockSpec((H, H), lambda i: (0, 0)),
            pl.BlockSpec((1, H), lambda i: (0, 0)),
        ],
        out_specs=pl.BlockSpec((BN, H), lambda i: (i, 0)),
        out_shape=jax.ShapeDtypeStruct((N, H), jnp.float32),
    )(h, sa, se, dega2, dege2, aW2, ab2, eW2, eb2, g2, bl2, cW, cb2)


def _make_conv(want_deg):
    mesh = plsc.VectorSubcoreMesh(core_axis_name="c", subcore_axis_name="s",
                                  num_cores=2, num_subcores=NSC)
    out_type = [jax.ShapeDtypeStruct((NCH, N, CH), jnp.float32)]
    if want_deg:
        out_type.append(jax.ShapeDtypeStruct((N,), jnp.float32))
    scratch = [
        pltpu.VMEM_SHARED((N, CH), jnp.float32),
        pltpu.VMEM_SHARED((N,), jnp.float32),
        pltpu.VMEM((3, 2, NSUB, SUB), jnp.int32),
        pltpu.VMEM((3, 2, B), jnp.float32),
        pltpu.VMEM((ED, CH), jnp.float32),
        pltpu.VMEM((2, B, CH), jnp.float32),
        pltpu.VMEM((2, B, CH), jnp.float32),
        pltpu.VMEM((2, B, CH), jnp.float32),
        pltpu.VMEM((SUB,), jnp.float32),
        pltpu.SemaphoreType.DMA,
        pltpu.SemaphoreType.DMA,
        pltpu.SemaphoreType.DMA,
        pltpu.SemaphoreType.DMA,
    ]

    def body(*args):
        pts = args[0:NCH]
        qts = args[NCH:2 * NCH]
        (ids, eab, wattr, z2d, z1d, ones_h, s_out) = args[2 * NCH:2 * NCH + 7]
        rest = args[2 * NCH + 7:]
        if want_deg:
            deg_out = rest[0]
            rest = rest[1:]
        (acc, deg_acc, idx_b, ea_b, w_v, pd, qs, tb, ones_v,
         sem_f, sem_g, sem_s0, sem_s1) = rest
        cid = lax.axis_index("c")
        sid = lax.axis_index("s")
        row0 = sid * RPT
        pltpu.sync_copy(ones_h, ones_v)

        def do_pass(chunk, with_deg):
            pt = pts[chunk]
            qt = qts[chunk]

            @pl.when(sid < NSC - 1)
            def _():
                pltpu.sync_copy(z2d, acc.at[pl.ds(row0, RPT)])

            @pl.when(sid == NSC - 1)
            def _():
                pltpu.sync_copy(z2d.at[pl.ds(0, RPT_LAST)],
                                acc.at[pl.ds(row0, RPT_LAST)])

            if with_deg:
                @pl.when(sid == 0)
                def _():
                    pltpu.sync_copy(z1d, deg_acc)
            pltpu.sync_copy(wattr.at[chunk], w_v)
            plsc.subcore_barrier()
            w0 = w_v[0, pl.ds(0, CH)]
            w1 = w_v[1, pl.ds(0, CH)]
            blk0 = sid * BLK_PER_TILE
            NB = BLK_PER_TILE

            def fetch_start(s3, j):
                pltpu.async_copy(ids.at[j], idx_b.at[s3], sem_f)
                pltpu.async_copy(eab.at[j], ea_b.at[s3], sem_f)

            def fetch_wait(s3):
                pltpu.make_async_copy(ids.at[0], idx_b.at[s3], sem_f).wait()
                pltpu.make_async_copy(eab.at[0], ea_b.at[s3], sem_f).wait()

            def gather_start(s2, s3):
                for k in range(NSUB):
                    pltpu.async_copy(pt.at[idx_b.at[s3, 0, k]],
                                     pd.at[s2].at[pl.ds(k * SUB, SUB)], sem_g)
                    pltpu.async_copy(qt.at[idx_b.at[s3, 1, k]],
                                     qs.at[s2].at[pl.ds(k * SUB, SUB)], sem_g)

            def gather_wait(s2, s3):
                for k in range(NSUB):
                    pltpu.make_async_copy(
                        pt.at[idx_b.at[s3, 0, k]],
                        pd.at[s2].at[pl.ds(k * SUB, SUB)], sem_g).wait()
                    pltpu.make_async_copy(
                        qt.at[idx_b.at[s3, 1, k]],
                        qs.at[s2].at[pl.ds(k * SUB, SUB)], sem_g).wait()

            def scat_start(s2, s3):
                for k in range(NSUB):
                    pltpu.async_copy(tb.at[s2].at[pl.ds(k * SUB, SUB)],
                                     acc.at[idx_b.at[s3, 0, k]],
                                     sem_s0, add=True)
                    if with_deg:
                        pltpu.async_copy(ones_v,
                                         deg_acc.at[idx_b.at[s3, 0, k]],
                                         sem_s0, add=True)

            def scat_wait(s2, s3):
                for k in range(NSUB):
                    pltpu.make_async_copy(
                        tb.at[s2].at[pl.ds(k * SUB, SUB)],
                        acc.at[idx_b.at[s3, 0, k]], sem_s0).wait()
                    if with_deg:
                        pltpu.make_async_copy(
                            ones_v, deg_acc.at[idx_b.at[s3, 0, k]],
                            sem_s0).wait()

            def compute(s2, s3):
                @pl.loop(0, (B + 15) // 16)
                def _grp(g):
                    e0 = jnp.minimum(g * 16, B - 16)
                    a0v = ea_b[s3, 0, pl.ds(e0, 16)]
                    a1v = ea_b[s3, 1, pl.ds(e0, 16)]
                    for i in range(16):
                        e = e0 + i
                        a0 = a0v[i]
                        a1 = a1v[i]
                        pv = pd[s2, e, pl.ds(0, CH)]
                        qv = qs[s2, e, pl.ds(0, CH)]
                        tb[s2, e, pl.ds(0, CH)] = jnp.maximum(
                            pv + qv + a0 * w0 + a1 * w1, 0.0)

            fetch_start(0, blk0)
            fetch_wait(0)
            gather_start(0, 0)

            @pl.when(NB > 1)
            def _():
                fetch_start(1, blk0 + 1)

            @pl.loop(0, NB)
            def _blk(jj):
                j = blk0 + jj
                s2 = jax.lax.rem(jj, 2)
                s3 = jax.lax.rem(jj, 3)
                s2n = jax.lax.rem(jj + 1, 2)
                s3n = jax.lax.rem(jj + 1, 3)
                s3nn = jax.lax.rem(jj + 2, 3)

                @pl.when(jj < NB - 1)
                def _():
                    fetch_wait(s3n)
                gather_wait(s2, s3)

                @pl.when(jj < NB - 1)
                def _():
                    gather_start(s2n, s3n)

                @pl.when(jj < NB - 2)
                def _():
                    fetch_start(s3nn, j + 2)

                @pl.when(jj >= 2)
                def _():
                    scat_wait(s2, s3)
                compute(s2, s3)
                scat_start(s2, s3)

            scat_wait(jax.lax.rem(NB - 2, 2), jax.lax.rem(NB - 2, 3))
            scat_wait(jax.lax.rem(NB - 1, 2), jax.lax.rem(NB - 1, 3))

            plsc.subcore_barrier()

            @pl.when(sid < NSC - 1)
            def _():
                pltpu.sync_copy(acc.at[pl.ds(row0, RPT)],
                                s_out.at[chunk].at[pl.ds(row0, RPT)])

            @pl.when(sid == NSC - 1)
            def _():
                pltpu.sync_copy(acc.at[pl.ds(row0, RPT_LAST)],
                                s_out.at[chunk].at[pl.ds(row0, RPT_LAST)])
            if with_deg:
                @pl.when(sid == 0)
                def _():
                    pltpu.sync_copy(deg_acc, deg_out)

        @pl.when(cid == 0)
        def _():
            for p in range(PPC):
                do_pass(p, want_deg and p == 0)

        @pl.when(cid == 1)
        def _():
            for p in range(PPC, NCH):
                do_pass(p, False)

    return pl.kernel(body, out_type=out_type, mesh=mesh,
                     scratch_types=scratch,
                     compiler_params=pltpu.CompilerParams(
                         use_tc_tiling_on_sc=False))


_conv_cache = {}


def _get_conv(want_deg):
    if want_deg not in _conv_cache:
        _conv_cache[want_deg] = _make_conv(want_deg)
    return _conv_cache[want_deg]


def kernel(x, ally_edge_index, ally_edge_attr, enc_edge_index, enc_edge_attr,
           W_in, b_in, ally_W1_0, ally_b1_0, ally_W2_0, ally_b2_0,
           enc_W1_0, enc_b1_0, enc_W2_0, enc_b2_0, ln_g_0, ln_b_0,
           comb_W_0, comb_b_0, ally_W1_1, ally_b1_1, ally_W2_1, ally_b2_1,
           enc_W1_1, enc_b1_1, enc_W2_1, enc_b2_1, ln_g_1, ln_b_1,
           comb_W_1, comb_b_1):
    f32 = jnp.float32
    conv_deg = _get_conv(True)
    conv_nodeg = _get_conv(False)

    xp = jnp.pad(x, ((0, 0), (0, H - D_IN)))
    Wp = jnp.pad(W_in, ((0, H - D_IN), (0, 0)))

    def edge_prep(ei, ea):
        dst3 = ei[1].reshape(NBLK, 1, NSUB, SUB)
        src3 = ei[0].reshape(NBLK, 1, NSUB, SUB)
        ids = jnp.concatenate([dst3, src3], axis=1)
        eab = jnp.stack([ea[:, 0].reshape(NBLK, B),
                         ea[:, 1].reshape(NBLK, B)], axis=1)
        return ids, eab

    a_ids, a_eab = edge_prep(ally_edge_index, ally_edge_attr)
    e_ids, e_eab = edge_prep(enc_edge_index, enc_edge_attr)

    def wcat_prep(aW1, ab1, eW1, eb1):
        Wcat = jnp.concatenate(
            [aW1[:H], aW1[H:2 * H], eW1[:H], eW1[H:2 * H]], axis=1)
        bcat = jnp.concatenate(
            [ab1, jnp.zeros((H,), f32), eb1, jnp.zeros((H,), f32)])
        wattr_a = aW1[2 * H:].reshape(ED, NCH, CH).transpose(1, 0, 2)
        wattr_e = eW1[2 * H:].reshape(ED, NCH, CH).transpose(1, 0, 2)
        return Wcat, bcat.reshape(1, -1), wattr_a, wattr_e

    Wcat0, bcat0, wattr_a0, wattr_e0 = wcat_prep(ally_W1_0, ally_b1_0,
                                                 enc_W1_0, enc_b1_0)
    Wcat1, bcat1, wattr_a1, wattr_e1 = wcat_prep(ally_W1_1, ally_b1_1,
                                                 enc_W1_1, enc_b1_1)

    z2d = jnp.zeros((RPT, CH), f32)
    z1d = jnp.zeros((N,), f32)
    ones_h = jnp.ones((SUB,), f32)

    def r2(v):
        return v.reshape(1, -1)

    outs = _input_tables_call(xp, Wp, r2(b_in), Wcat0, bcat0)
    h = outs[0]
    tabs = outs[1:]
    ap, aq = tabs[0:NCH], tabs[NCH:2 * NCH]
    ep, eq = tabs[2 * NCH:3 * NCH], tabs[3 * NCH:4 * NCH]

    sa, dega = conv_deg(*ap, *aq, a_ids, a_eab,
                        wattr_a0, z2d, z1d, ones_h)
    se, dege = conv_deg(*ep, *eq, e_ids, e_eab,
                        wattr_e0, z2d, z1d, ones_h)

    dega2 = dega.reshape(N // BN, 1, BN)
    dege2 = dege.reshape(N // BN, 1, BN)

    h = _combine_call(h, sa, se, dega2, dege2,
                      ally_W2_0, r2(ally_b2_0), enc_W2_0, r2(enc_b2_0),
                      r2(ln_g_0), r2(ln_b_0), comb_W_0, r2(comb_b_0))

    tabs = _tables_call(h, Wcat1, bcat1)
    ap, aq = tabs[0:NCH], tabs[NCH:2 * NCH]
    ep, eq = tabs[2 * NCH:3 * NCH], tabs[3 * NCH:4 * NCH]

    (sa,) = conv_nodeg(*ap, *aq, a_ids, a_eab,
                       wattr_a1, z2d, z1d, ones_h)
    (se,) = conv_nodeg(*ep, *eq, e_ids, e_eab,
                       wattr_e1, z2d, z1d, ones_h)

    h = _combine_call(h, sa, se, dega2, dege2,
                      ally_W2_1, r2(ally_b2_1), enc_W2_1, r2(enc_b2_1),
                      r2(ln_g_1), r2(ln_b_1), comb_W_1, r2(comb_b_1))
    return h


# speedup vs baseline: 2.5181x; 1.1537x over previous
import jax
import jax.numpy as jnp
from jax import lax
from jax.experimental import pallas as pl
from jax.experimental.pallas import tpu as pltpu
from jax.experimental.pallas import tpu_sc as plsc

N = 50000
E = 800000
D_IN = 39
H = 128
ED = 2

BN = 2000
CH = 16
NCH = H // CH
PPC = NCH // 2
B = 1000
SUB = 125
NSUB = B // SUB
NBLK = E // B
NSC = 16
BLK_PER_TILE = NBLK // NSC
RPT = 3128
RPT_LAST = N - 15 * RPT


def _input_tables_call(xp, Wp, b_in2, Wcat, bcat2):
    BT = 1000
    grid = (N // BT,)
    outs = ([jax.ShapeDtypeStruct((N, H), jnp.float32)] +
            [jax.ShapeDtypeStruct((N, CH), jnp.float32) for _ in range(4 * NCH)])

    def body(x_ref, wp_ref, b_ref, wc_ref, bc_ref, h_ref, *tab_refs):
        hb = jnp.maximum(
            jnp.dot(x_ref[...], wp_ref[...], preferred_element_type=jnp.float32)
            + b_ref[...], 0.0)
        h_ref[...] = hb
        full = jnp.dot(hb, wc_ref[...], preferred_element_type=jnp.float32) + bc_ref[...]
        for t in range(4 * NCH):
            tab_refs[t][...] = full[:, t * CH:(t + 1) * CH]

    return pl.pallas_call(
        body,
        grid=grid,
        in_specs=[
            pl.BlockSpec((BT, H), lambda i: (i, 0)),
            pl.BlockSpec((H, H), lambda i: (0, 0)),
            pl.BlockSpec((1, H), lambda i: (0, 0)),
            pl.BlockSpec((H, 4 * NCH * CH), lambda i: (0, 0)),
            pl.BlockSpec((1, 4 * NCH * CH), lambda i: (0, 0)),
        ],
        out_specs=([pl.BlockSpec((BT, H), lambda i: (i, 0))] +
                   [pl.BlockSpec((BT, CH), lambda i: (i, 0))] * (4 * NCH)),
        out_shape=outs,
    )(xp, Wp, b_in2, Wcat, bcat2)


def _tables_call(h, Wcat, bcat2):
    BT = 1000
    grid = (N // BT,)
    outs = [jax.ShapeDtypeStruct((N, CH), jnp.float32) for _ in range(4 * NCH)]

    def body(h_ref, wc_ref, bc_ref, *tab_refs):
        full = jnp.dot(h_ref[...], wc_ref[...],
                       preferred_element_type=jnp.float32) + bc_ref[...]
        for t in range(4 * NCH):
            tab_refs[t][...] = full[:, t * CH:(t + 1) * CH]

    return pl.pallas_call(
        body,
        grid=grid,
        in_specs=[
            pl.BlockSpec((BT, H), lambda i: (i, 0)),
            pl.BlockSpec((H, 4 * NCH * CH), lambda i: (0, 0)),
            pl.BlockSpec((1, 4 * NCH * CH), lambda i: (0, 0)),
        ],
        out_specs=[pl.BlockSpec((BT, CH), lambda i: (i, 0))] * (4 * NCH),
        out_shape=outs,
    )(h, Wcat, bcat2)


def _combine_call(h, sa, se, dega2, dege2, aW2, ab2, eW2, eb2, g2, bl2, cW, cb2):
    grid = (N // BN,)

    def body(h_ref, sa_ref, se_ref, da_ref, de_ref, aw_ref, ab_ref,
             ew_ref, eb_ref, g_ref, bl_ref, cw_ref, cb_ref, o_ref):
        sa_full = jnp.concatenate([sa_ref[c] for c in range(NCH)], axis=-1)
        se_full = jnp.concatenate([se_ref[c] for c in range(NCH)], axis=-1)
        h1 = (h_ref[...]
              + jnp.dot(sa_full, aw_ref[...], preferred_element_type=jnp.float32)
              + da_ref[0, 0][:, None] * ab_ref[...]
              + jnp.dot(se_full, ew_ref[...], preferred_element_type=jnp.float32)
              + de_ref[0, 0][:, None] * eb_ref[...])
        mu = jnp.mean(h1, axis=-1, keepdims=True)
        var = jnp.mean((h1 - mu) ** 2, axis=-1, keepdims=True)
        t = (h1 - mu) / jnp.sqrt(var + 1e-5) * g_ref[...] + bl_ref[...]
        o_ref[...] = t + jnp.maximum(
            jnp.dot(t, cw_ref[...], preferred_element_type=jnp.float32)
            + cb_ref[...], 0.0)

    return pl.pallas_call(
        body,
        grid=grid,
        in_specs=[
            pl.BlockSpec((BN, H), lambda i: (i, 0)),
            pl.BlockSpec((NCH, BN, CH), lambda i: (0, i, 0)),
            pl.BlockSpec((NCH, BN, CH), lambda i: (0, i, 0)),
            pl.BlockSpec((1, 1, BN), lambda i: (i, 0, 0)),
            pl.BlockSpec((1, 1, BN), lambda i: (i, 0, 0)),
            pl.BlockSpec((H, H), lambda i: (0, 0)),
            pl.BlockSpec((1, H), lambda i: (0, 0)),
            pl.BlockSpec((H, H), lambda i: (0, 0)),
            pl.BlockSpec((1, H), lambda i: (0, 0)),
            pl.BlockSpec((1, H), lambda i: (0, 0)),
            pl.BlockSpec((1, H), lambda i: (0, 0)),
            pl.BlockSpec((H, H), lambda i: (0, 0)),
            pl.BlockSpec((1, H), lambda i: (0, 0)),
        ],
        out_specs=pl.BlockSpec((BN, H), lambda i: (i, 0)),
        out_shape=jax.ShapeDtypeStruct((N, H), jnp.float32),
    )(h, sa, se, dega2, dege2, aW2, ab2, eW2, eb2, g2, bl2, cW, cb2)


def _make_conv(want_deg):
    mesh = plsc.VectorSubcoreMesh(core_axis_name="c", subcore_axis_name="s",
                                  num_cores=2, num_subcores=NSC)
    out_type = [jax.ShapeDtypeStruct((NCH, N, CH), jnp.float32)]
    if want_deg:
        out_type.append(jax.ShapeDtypeStruct((N,), jnp.float32))
    scratch = [
        pltpu.VMEM_SHARED((N, CH), jnp.float32),
        pltpu.VMEM_SHARED((N,), jnp.float32),
        pltpu.VMEM((3, 2, B), jnp.int32),
        pltpu.VMEM((3, 2, B), jnp.float32),
        pltpu.VMEM((ED, CH), jnp.float32),
        pltpu.VMEM((2, B, CH), jnp.float32),
        pltpu.VMEM((B,), jnp.float32),
        pltpu.SemaphoreType.DMA,
        pltpu.SemaphoreType.DMA,
        pltpu.SemaphoreType.DMA,
    ]

    def body(*args):
        pts = args[0:NCH]
        qts = args[NCH:2 * NCH]
        (ids, eab, wattr, z2d, z1d, ones_h, s_out) = args[2 * NCH:2 * NCH + 7]
        rest = args[2 * NCH + 7:]
        if want_deg:
            deg_out = rest[0]
            rest = rest[1:]
        (acc, deg_acc, idx_b, ea_b, w_v, tb, ones_v,
         sem_f, sem_g, sem_s) = rest
        cid = lax.axis_index("c")
        sid = lax.axis_index("s")
        row0 = sid * RPT
        pltpu.sync_copy(ones_h, ones_v)

        def do_pass(chunk, with_deg):
            pt = pts[chunk]
            qt = qts[chunk]

            @pl.when(sid < NSC - 1)
            def _():
                pltpu.sync_copy(z2d, acc.at[pl.ds(row0, RPT)])

            @pl.when(sid == NSC - 1)
            def _():
                pltpu.sync_copy(z2d.at[pl.ds(0, RPT_LAST)],
                                acc.at[pl.ds(row0, RPT_LAST)])

            if with_deg:
                @pl.when(sid == 0)
                def _():
                    pltpu.sync_copy(z1d, deg_acc)
            pltpu.sync_copy(wattr.at[chunk], w_v)
            plsc.subcore_barrier()
            w0 = w_v[0, pl.ds(0, CH)]
            w1 = w_v[1, pl.ds(0, CH)]
            blk0 = sid * BLK_PER_TILE
            NB = BLK_PER_TILE

            def fetch_start(s3, j):
                pltpu.async_copy(ids.at[j], idx_b.at[s3], sem_f)
                pltpu.async_copy(eab.at[j], ea_b.at[s3], sem_f)

            def fetch_wait(s3):
                pltpu.make_async_copy(ids.at[0], idx_b.at[s3], sem_f).wait()
                pltpu.make_async_copy(eab.at[0], ea_b.at[s3], sem_f).wait()

            def gather_start(s2, s3):
                pltpu.async_copy(pt.at[idx_b.at[s3, 0]], tb.at[s2], sem_g,
                                 add=True)
                pltpu.async_copy(qt.at[idx_b.at[s3, 1]], tb.at[s2], sem_g,
                                 add=True)

            def gather_wait(s2, s3):
                pltpu.make_async_copy(pt.at[idx_b.at[s3, 0]], tb.at[s2],
                                      sem_g).wait()
                pltpu.make_async_copy(qt.at[idx_b.at[s3, 1]], tb.at[s2],
                                      sem_g).wait()

            def scat_start(s2, s3):
                pltpu.async_copy(tb.at[s2], acc.at[idx_b.at[s3, 0]], sem_s,
                                 add=True)
                if with_deg:
                    pltpu.async_copy(ones_v, deg_acc.at[idx_b.at[s3, 0]],
                                     sem_s, add=True)

            def scat_wait(s2, s3):
                pltpu.make_async_copy(tb.at[s2], acc.at[idx_b.at[s3, 0]],
                                      sem_s).wait()
                if with_deg:
                    pltpu.make_async_copy(ones_v, deg_acc.at[idx_b.at[s3, 0]],
                                          sem_s).wait()

            def attr_fill(s2, s3):
                @pl.loop(0, (B + 15) // 16)
                def _grp(g):
                    e0 = jnp.minimum(g * 16, B - 16)
                    a0v = ea_b[s3, 0, pl.ds(e0, 16)]
                    a1v = ea_b[s3, 1, pl.ds(e0, 16)]
                    for i in range(16):
                        tb[s2, e0 + i, pl.ds(0, CH)] = (
                            a0v[i] * w0 + a1v[i] * w1)

            def relu_inplace(s2):
                @pl.loop(0, B)
                def _r(i):
                    v = tb[s2, i, pl.ds(0, CH)]
                    tb[s2, i, pl.ds(0, CH)] = jnp.maximum(v, 0.0)

            fetch_start(0, blk0)
            fetch_wait(0)
            attr_fill(0, 0)
            gather_start(0, 0)

            @pl.when(NB > 1)
            def _():
                fetch_start(1, blk0 + 1)

            @pl.loop(0, NB)
            def _blk(jj):
                j = blk0 + jj
                s2 = jax.lax.rem(jj, 2)
                s3 = jax.lax.rem(jj, 3)
                s2n = jax.lax.rem(jj + 1, 2)
                s3n = jax.lax.rem(jj + 1, 3)
                s3nn = jax.lax.rem(jj + 2, 3)

                @pl.when(jj < NB - 1)
                def _():
                    fetch_wait(s3n)

                    @pl.when(jj >= 1)
                    def _():
                        scat_wait(s2n, s3n)
                    attr_fill(s2n, s3n)
                    gather_start(s2n, s3n)

                @pl.when(jj < NB - 2)
                def _():
                    fetch_start(s3nn, j + 2)

                gather_wait(s2, s3)
                relu_inplace(s2)
                scat_start(s2, s3)

            @pl.when(NB > 1)
            def _():
                scat_wait(jax.lax.rem(NB - 2, 2), jax.lax.rem(NB - 2, 3))
            scat_wait(jax.lax.rem(NB - 1, 2), jax.lax.rem(NB - 1, 3))

            plsc.subcore_barrier()

            @pl.when(sid < NSC - 1)
            def _():
                pltpu.sync_copy(acc.at[pl.ds(row0, RPT)],
                                s_out.at[chunk].at[pl.ds(row0, RPT)])

            @pl.when(sid == NSC - 1)
            def _():
                pltpu.sync_copy(acc.at[pl.ds(row0, RPT_LAST)],
                                s_out.at[chunk].at[pl.ds(row0, RPT_LAST)])
            if with_deg:
                @pl.when(sid == 0)
                def _():
                    pltpu.sync_copy(deg_acc, deg_out)

        @pl.when(cid == 0)
        def _():
            for p in range(PPC):
                do_pass(p, want_deg and p == 0)

        @pl.when(cid == 1)
        def _():
            for p in range(PPC, NCH):
                do_pass(p, False)

    return pl.kernel(body, out_type=out_type, mesh=mesh,
                     scratch_types=scratch,
                     compiler_params=pltpu.CompilerParams(
                         use_tc_tiling_on_sc=False))


_conv_cache = {}


def _get_conv(want_deg):
    if want_deg not in _conv_cache:
        _conv_cache[want_deg] = _make_conv(want_deg)
    return _conv_cache[want_deg]


def kernel(x, ally_edge_index, ally_edge_attr, enc_edge_index, enc_edge_attr,
           W_in, b_in, ally_W1_0, ally_b1_0, ally_W2_0, ally_b2_0,
           enc_W1_0, enc_b1_0, enc_W2_0, enc_b2_0, ln_g_0, ln_b_0,
           comb_W_0, comb_b_0, ally_W1_1, ally_b1_1, ally_W2_1, ally_b2_1,
           enc_W1_1, enc_b1_1, enc_W2_1, enc_b2_1, ln_g_1, ln_b_1,
           comb_W_1, comb_b_1):
    f32 = jnp.float32
    conv_deg = _get_conv(True)
    conv_nodeg = _get_conv(False)

    xp = jnp.pad(x, ((0, 0), (0, H - D_IN)))
    Wp = jnp.pad(W_in, ((0, H - D_IN), (0, 0)))

    def edge_prep(ei, ea):
        dst3 = ei[1].reshape(NBLK, 1, B)
        src3 = ei[0].reshape(NBLK, 1, B)
        ids = jnp.concatenate([dst3, src3], axis=1)
        eab = jnp.stack([ea[:, 0].reshape(NBLK, B),
                         ea[:, 1].reshape(NBLK, B)], axis=1)
        return ids, eab

    a_ids, a_eab = edge_prep(ally_edge_index, ally_edge_attr)
    e_ids, e_eab = edge_prep(enc_edge_index, enc_edge_attr)

    def wcat_prep(aW1, ab1, eW1, eb1):
        Wcat = jnp.concatenate(
            [aW1[:H], aW1[H:2 * H], eW1[:H], eW1[H:2 * H]], axis=1)
        bcat = jnp.concatenate(
            [ab1, jnp.zeros((H,), f32), eb1, jnp.zeros((H,), f32)])
        wattr_a = aW1[2 * H:].reshape(ED, NCH, CH).transpose(1, 0, 2)
        wattr_e = eW1[2 * H:].reshape(ED, NCH, CH).transpose(1, 0, 2)
        return Wcat, bcat.reshape(1, -1), wattr_a, wattr_e

    Wcat0, bcat0, wattr_a0, wattr_e0 = wcat_prep(ally_W1_0, ally_b1_0,
                                                 enc_W1_0, enc_b1_0)
    Wcat1, bcat1, wattr_a1, wattr_e1 = wcat_prep(ally_W1_1, ally_b1_1,
                                                 enc_W1_1, enc_b1_1)

    z2d = jnp.zeros((RPT, CH), f32)
    z1d = jnp.zeros((N,), f32)
    ones_h = jnp.ones((B,), f32)

    def r2(v):
        return v.reshape(1, -1)

    outs = _input_tables_call(xp, Wp, r2(b_in), Wcat0, bcat0)
    h = outs[0]
    tabs = outs[1:]
    ap, aq = tabs[0:NCH], tabs[NCH:2 * NCH]
    ep, eq = tabs[2 * NCH:3 * NCH], tabs[3 * NCH:4 * NCH]

    sa, dega = conv_deg(*ap, *aq, a_ids, a_eab,
                        wattr_a0, z2d, z1d, ones_h)
    se, dege = conv_deg(*ep, *eq, e_ids, e_eab,
                        wattr_e0, z2d, z1d, ones_h)

    dega2 = dega.reshape(N // BN, 1, BN)
    dege2 = dege.reshape(N // BN, 1, BN)

    h = _combine_call(h, sa, se, dega2, dege2,
                      ally_W2_0, r2(ally_b2_0), enc_W2_0, r2(enc_b2_0),
                      r2(ln_g_0), r2(ln_b_0), comb_W_0, r2(comb_b_0))

    tabs = _tables_call(h, Wcat1, bcat1)
    ap, aq = tabs[0:NCH], tabs[NCH:2 * NCH]
    ep, eq = tabs[2 * NCH:3 * NCH], tabs[3 * NCH:4 * NCH]

    (sa,) = conv_nodeg(*ap, *aq, a_ids, a_eab,
                       wattr_a1, z2d, z1d, ones_h)
    (se,) = conv_nodeg(*ep, *eq, e_ids, e_eab,
                       wattr_e1, z2d, z1d, ones_h)

    h = _combine_call(h, sa, se, dega2, dege2,
                      ally_W2_1, r2(ally_b2_1), enc_W2_1, r2(enc_b2_1),
                      r2(ln_g_1), r2(ln_b_1), comb_W_1, r2(comb_b_1))
    return h


# speedup vs baseline: 4.0615x; 1.6129x over previous
import jax
import jax.numpy as jnp
from jax import lax
from jax.experimental import pallas as pl
from jax.experimental.pallas import tpu as pltpu
from jax.experimental.pallas import tpu_sc as plsc

N = 50000
E = 800000
D_IN = 39
H = 128
ED = 2

BN = 2000
CH = 16
NCH = H // CH
PPC = NCH // 2
B = 1000
SUB = 125
NSUB = B // SUB
NBLK = E // B
NSC = 16
BLK_PER_TILE = NBLK // NSC
RPT = 3128
RPT_LAST = N - 15 * RPT


def _input_tables_call(xp, Wp, b_in2, Wcat, bcat2):
    BT = 1000
    grid = (N // BT,)
    outs = ([jax.ShapeDtypeStruct((N, H), jnp.float32)] +
            [jax.ShapeDtypeStruct((N, CH), jnp.float32) for _ in range(4 * NCH)])

    def body(x_ref, wp_ref, b_ref, wc_ref, bc_ref, h_ref, *tab_refs):
        hb = jnp.maximum(
            jnp.dot(x_ref[...], wp_ref[...], preferred_element_type=jnp.float32)
            + b_ref[...], 0.0)
        h_ref[...] = hb
        full = jnp.dot(hb, wc_ref[...], preferred_element_type=jnp.float32) + bc_ref[...]
        for t in range(4 * NCH):
            tab_refs[t][...] = full[:, t * CH:(t + 1) * CH]

    return pl.pallas_call(
        body,
        grid=grid,
        in_specs=[
            pl.BlockSpec((BT, H), lambda i: (i, 0)),
            pl.BlockSpec((H, H), lambda i: (0, 0)),
            pl.BlockSpec((1, H), lambda i: (0, 0)),
            pl.BlockSpec((H, 4 * NCH * CH), lambda i: (0, 0)),
            pl.BlockSpec((1, 4 * NCH * CH), lambda i: (0, 0)),
        ],
        out_specs=([pl.BlockSpec((BT, H), lambda i: (i, 0))] +
                   [pl.BlockSpec((BT, CH), lambda i: (i, 0))] * (4 * NCH)),
        out_shape=outs,
    )(xp, Wp, b_in2, Wcat, bcat2)


def _tables_call(h, Wcat, bcat2):
    BT = 1000
    grid = (N // BT,)
    outs = [jax.ShapeDtypeStruct((N, CH), jnp.float32) for _ in range(4 * NCH)]

    def body(h_ref, wc_ref, bc_ref, *tab_refs):
        full = jnp.dot(h_ref[...], wc_ref[...],
                       preferred_element_type=jnp.float32) + bc_ref[...]
        for t in range(4 * NCH):
            tab_refs[t][...] = full[:, t * CH:(t + 1) * CH]

    return pl.pallas_call(
        body,
        grid=grid,
        in_specs=[
            pl.BlockSpec((BT, H), lambda i: (i, 0)),
            pl.BlockSpec((H, 4 * NCH * CH), lambda i: (0, 0)),
            pl.BlockSpec((1, 4 * NCH * CH), lambda i: (0, 0)),
        ],
        out_specs=[pl.BlockSpec((BT, CH), lambda i: (i, 0))] * (4 * NCH),
        out_shape=outs,
    )(h, Wcat, bcat2)


def _combine_call(h, sa, se, dega2, dege2, aW2, ab2, eW2, eb2, g2, bl2, cW, cb2):
    grid = (N // BN,)

    def body(h_ref, sa_ref, se_ref, da_ref, de_ref, aw_ref, ab_ref,
             ew_ref, eb_ref, g_ref, bl_ref, cw_ref, cb_ref, o_ref):
        sa_full = jnp.concatenate([sa_ref[c] for c in range(NCH)], axis=-1)
        se_full = jnp.concatenate([se_ref[c] for c in range(NCH)], axis=-1)
        h1 = (h_ref[...]
              + jnp.dot(sa_full, aw_ref[...], preferred_element_type=jnp.float32)
              + da_ref[0, 0][:, None] * ab_ref[...]
              + jnp.dot(se_full, ew_ref[...], preferred_element_type=jnp.float32)
              + de_ref[0, 0][:, None] * eb_ref[...])
        mu = jnp.mean(h1, axis=-1, keepdims=True)
        var = jnp.mean((h1 - mu) ** 2, axis=-1, keepdims=True)
        t = (h1 - mu) / jnp.sqrt(var + 1e-5) * g_ref[...] + bl_ref[...]
        o_ref[...] = t + jnp.maximum(
            jnp.dot(t, cw_ref[...], preferred_element_type=jnp.float32)
            + cb_ref[...], 0.0)

    return pl.pallas_call(
        body,
        grid=grid,
        in_specs=[
            pl.BlockSpec((BN, H), lambda i: (i, 0)),
            pl.BlockSpec((NCH, BN, CH), lambda i: (0, i, 0)),
            pl.BlockSpec((NCH, BN, CH), lambda i: (0, i, 0)),
            pl.BlockSpec((1, 1, BN), lambda i: (i, 0, 0)),
            pl.BlockSpec((1, 1, BN), lambda i: (i, 0, 0)),
            pl.BlockSpec((H, H), lambda i: (0, 0)),
            pl.BlockSpec((1, H), lambda i: (0, 0)),
            pl.BlockSpec((H, H), lambda i: (0, 0)),
            pl.BlockSpec((1, H), lambda i: (0, 0)),
            pl.BlockSpec((1, H), lambda i: (0, 0)),
            pl.BlockSpec((1, H), lambda i: (0, 0)),
            pl.BlockSpec((H, H), lambda i: (0, 0)),
            pl.BlockSpec((1, H), lambda i: (0, 0)),
        ],
        out_specs=pl.BlockSpec((BN, H), lambda i: (i, 0)),
        out_shape=jax.ShapeDtypeStruct((N, H), jnp.float32),
    )(h, sa, se, dega2, dege2, aW2, ab2, eW2, eb2, g2, bl2, cW, cb2)


def _make_conv(want_deg):
    mesh = plsc.VectorSubcoreMesh(core_axis_name="c", subcore_axis_name="s",
                                  num_cores=2, num_subcores=NSC)
    out_type = [jax.ShapeDtypeStruct((NCH, N, CH), jnp.float32)]
    if want_deg:
        out_type.append(jax.ShapeDtypeStruct((N,), jnp.float32))
    scratch = [
        pltpu.VMEM_SHARED((N, CH), jnp.float32),
        pltpu.VMEM_SHARED((N,), jnp.float32),
        pltpu.VMEM((3, 2, B), jnp.int32),
        pltpu.VMEM((3, 2, B), jnp.float32),
        pltpu.VMEM((ED, CH), jnp.float32),
        pltpu.VMEM((2, B, CH), jnp.float32),
        pltpu.VMEM((B,), jnp.float32),
        pltpu.SemaphoreType.DMA,
        pltpu.SemaphoreType.DMA,
        pltpu.SemaphoreType.DMA,
    ]

    def body(*args):
        pts = args[0:NCH]
        qts = args[NCH:2 * NCH]
        (ids, eab, wattr, z2d, z1d, ones_h, s_out) = args[2 * NCH:2 * NCH + 7]
        rest = args[2 * NCH + 7:]
        if want_deg:
            deg_out = rest[0]
            rest = rest[1:]
        (acc, deg_acc, idx_b, ea_b, w_v, tb, ones_v,
         sem_f, sem_g, sem_s) = rest
        cid = lax.axis_index("c")
        sid = lax.axis_index("s")
        row0 = sid * RPT
        pltpu.sync_copy(ones_h, ones_v)

        def do_pass(chunk, with_deg):
            pt = pts[chunk]
            qt = qts[chunk]

            @pl.when(sid < NSC - 1)
            def _():
                pltpu.sync_copy(z2d, acc.at[pl.ds(row0, RPT)])

            @pl.when(sid == NSC - 1)
            def _():
                pltpu.sync_copy(z2d.at[pl.ds(0, RPT_LAST)],
                                acc.at[pl.ds(row0, RPT_LAST)])

            if with_deg:
                @pl.when(sid == 0)
                def _():
                    pltpu.sync_copy(z1d, deg_acc)
            pltpu.sync_copy(wattr.at[chunk], w_v)
            plsc.subcore_barrier()
            w0 = w_v[0, pl.ds(0, CH)]
            w1 = w_v[1, pl.ds(0, CH)]
            blk0 = sid * BLK_PER_TILE
            NB = BLK_PER_TILE

            def fetch_start(s3, j):
                pltpu.async_copy(ids.at[j], idx_b.at[s3], sem_f)
                pltpu.async_copy(eab.at[j], ea_b.at[s3], sem_f)

            def fetch_wait(s3):
                pltpu.make_async_copy(ids.at[0], idx_b.at[s3], sem_f).wait()
                pltpu.make_async_copy(eab.at[0], ea_b.at[s3], sem_f).wait()

            def gather_start(s2, s3):
                pltpu.async_copy(pt.at[idx_b.at[s3, 0]], tb.at[s2], sem_g,
                                 add=True)
                pltpu.async_copy(qt.at[idx_b.at[s3, 1]], tb.at[s2], sem_g,
                                 add=True)

            def gather_wait(s2, s3):
                pltpu.make_async_copy(pt.at[idx_b.at[s3, 0]], tb.at[s2],
                                      sem_g).wait()
                pltpu.make_async_copy(qt.at[idx_b.at[s3, 1]], tb.at[s2],
                                      sem_g).wait()

            def scat_start(s2, s3):
                pltpu.async_copy(tb.at[s2], acc.at[idx_b.at[s3, 0]], sem_s,
                                 add=True)
                if with_deg:
                    pltpu.async_copy(ones_v, deg_acc.at[idx_b.at[s3, 0]],
                                     sem_s, add=True)

            def scat_wait(s2, s3):
                pltpu.make_async_copy(tb.at[s2], acc.at[idx_b.at[s3, 0]],
                                      sem_s).wait()
                if with_deg:
                    pltpu.make_async_copy(ones_v, deg_acc.at[idx_b.at[s3, 0]],
                                          sem_s).wait()

            def attr_fill(s2, s3):
                @pl.loop(0, (B + 15) // 16, unroll=2)
                def _grp(g):
                    e0 = jnp.minimum(g * 16, B - 16)
                    a0v = ea_b[s3, 0, pl.ds(e0, 16)]
                    a1v = ea_b[s3, 1, pl.ds(e0, 16)]
                    for i in range(16):
                        tb[s2, e0 + i, pl.ds(0, CH)] = (
                            a0v[i] * w0 + a1v[i] * w1)

            def relu_inplace(s2):
                @pl.loop(0, B, unroll=8)
                def _r(i):
                    v = tb[s2, i, pl.ds(0, CH)]
                    tb[s2, i, pl.ds(0, CH)] = jnp.maximum(v, 0.0)

            fetch_start(0, blk0)
            fetch_wait(0)
            attr_fill(0, 0)
            gather_start(0, 0)

            @pl.when(NB > 1)
            def _():
                fetch_start(1, blk0 + 1)

            @pl.loop(0, NB)
            def _blk(jj):
                j = blk0 + jj
                s2 = jax.lax.rem(jj, 2)
                s3 = jax.lax.rem(jj, 3)
                s2n = jax.lax.rem(jj + 1, 2)
                s3n = jax.lax.rem(jj + 1, 3)
                s3nn = jax.lax.rem(jj + 2, 3)

                @pl.when(jj < NB - 1)
                def _():
                    fetch_wait(s3n)

                    @pl.when(jj >= 1)
                    def _():
                        scat_wait(s2n, s3n)
                    attr_fill(s2n, s3n)
                    gather_start(s2n, s3n)

                @pl.when(jj < NB - 2)
                def _():
                    fetch_start(s3nn, j + 2)

                gather_wait(s2, s3)
                relu_inplace(s2)
                scat_start(s2, s3)

            @pl.when(NB > 1)
            def _():
                scat_wait(jax.lax.rem(NB - 2, 2), jax.lax.rem(NB - 2, 3))
            scat_wait(jax.lax.rem(NB - 1, 2), jax.lax.rem(NB - 1, 3))

            plsc.subcore_barrier()

            @pl.when(sid < NSC - 1)
            def _():
                pltpu.sync_copy(acc.at[pl.ds(row0, RPT)],
                                s_out.at[chunk].at[pl.ds(row0, RPT)])

            @pl.when(sid == NSC - 1)
            def _():
                pltpu.sync_copy(acc.at[pl.ds(row0, RPT_LAST)],
                                s_out.at[chunk].at[pl.ds(row0, RPT_LAST)])
            if with_deg:
                @pl.when(sid == 0)
                def _():
                    pltpu.sync_copy(deg_acc, deg_out)

        @pl.when(cid == 0)
        def _():
            for p in range(PPC):
                do_pass(p, want_deg and p == 0)

        @pl.when(cid == 1)
        def _():
            for p in range(PPC, NCH):
                do_pass(p, False)

    return pl.kernel(body, out_type=out_type, mesh=mesh,
                     scratch_types=scratch,
                     compiler_params=pltpu.CompilerParams(
                         use_tc_tiling_on_sc=False))


_conv_cache = {}


def _get_conv(want_deg):
    if want_deg not in _conv_cache:
        _conv_cache[want_deg] = _make_conv(want_deg)
    return _conv_cache[want_deg]


def kernel(x, ally_edge_index, ally_edge_attr, enc_edge_index, enc_edge_attr,
           W_in, b_in, ally_W1_0, ally_b1_0, ally_W2_0, ally_b2_0,
           enc_W1_0, enc_b1_0, enc_W2_0, enc_b2_0, ln_g_0, ln_b_0,
           comb_W_0, comb_b_0, ally_W1_1, ally_b1_1, ally_W2_1, ally_b2_1,
           enc_W1_1, enc_b1_1, enc_W2_1, enc_b2_1, ln_g_1, ln_b_1,
           comb_W_1, comb_b_1):
    f32 = jnp.float32
    conv_deg = _get_conv(True)
    conv_nodeg = _get_conv(False)

    xp = jnp.pad(x, ((0, 0), (0, H - D_IN)))
    Wp = jnp.pad(W_in, ((0, H - D_IN), (0, 0)))

    def edge_prep(ei, ea):
        dst3 = ei[1].reshape(NBLK, 1, B)
        src3 = ei[0].reshape(NBLK, 1, B)
        ids = jnp.concatenate([dst3, src3], axis=1)
        eab = jnp.stack([ea[:, 0].reshape(NBLK, B),
                         ea[:, 1].reshape(NBLK, B)], axis=1)
        return ids, eab

    a_ids, a_eab = edge_prep(ally_edge_index, ally_edge_attr)
    e_ids, e_eab = edge_prep(enc_edge_index, enc_edge_attr)

    def wcat_prep(aW1, ab1, eW1, eb1):
        Wcat = jnp.concatenate(
            [aW1[:H], aW1[H:2 * H], eW1[:H], eW1[H:2 * H]], axis=1)
        bcat = jnp.concatenate(
            [ab1, jnp.zeros((H,), f32), eb1, jnp.zeros((H,), f32)])
        wattr_a = aW1[2 * H:].reshape(ED, NCH, CH).transpose(1, 0, 2)
        wattr_e = eW1[2 * H:].reshape(ED, NCH, CH).transpose(1, 0, 2)
        return Wcat, bcat.reshape(1, -1), wattr_a, wattr_e

    Wcat0, bcat0, wattr_a0, wattr_e0 = wcat_prep(ally_W1_0, ally_b1_0,
                                                 enc_W1_0, enc_b1_0)
    Wcat1, bcat1, wattr_a1, wattr_e1 = wcat_prep(ally_W1_1, ally_b1_1,
                                                 enc_W1_1, enc_b1_1)

    z2d = jnp.zeros((RPT, CH), f32)
    z1d = jnp.zeros((N,), f32)
    ones_h = jnp.ones((B,), f32)

    def r2(v):
        return v.reshape(1, -1)

    outs = _input_tables_call(xp, Wp, r2(b_in), Wcat0, bcat0)
    h = outs[0]
    tabs = outs[1:]
    ap, aq = tabs[0:NCH], tabs[NCH:2 * NCH]
    ep, eq = tabs[2 * NCH:3 * NCH], tabs[3 * NCH:4 * NCH]

    sa, dega = conv_deg(*ap, *aq, a_ids, a_eab,
                        wattr_a0, z2d, z1d, ones_h)
    se, dege = conv_deg(*ep, *eq, e_ids, e_eab,
                        wattr_e0, z2d, z1d, ones_h)

    dega2 = dega.reshape(N // BN, 1, BN)
    dege2 = dege.reshape(N // BN, 1, BN)

    h = _combine_call(h, sa, se, dega2, dege2,
                      ally_W2_0, r2(ally_b2_0), enc_W2_0, r2(enc_b2_0),
                      r2(ln_g_0), r2(ln_b_0), comb_W_0, r2(comb_b_0))

    tabs = _tables_call(h, Wcat1, bcat1)
    ap, aq = tabs[0:NCH], tabs[NCH:2 * NCH]
    ep, eq = tabs[2 * NCH:3 * NCH], tabs[3 * NCH:4 * NCH]

    (sa,) = conv_nodeg(*ap, *aq, a_ids, a_eab,
                       wattr_a1, z2d, z1d, ones_h)
    (se,) = conv_nodeg(*ep, *eq, e_ids, e_eab,
                       wattr_e1, z2d, z1d, ones_h)

    h = _combine_call(h, sa, se, dega2, dege2,
                      ally_W2_1, r2(ally_b2_1), enc_W2_1, r2(enc_b2_1),
                      r2(ln_g_1), r2(ln_b_1), comb_W_1, r2(comb_b_1))
    return h


# speedup vs baseline: 5.4339x; 1.3379x over previous
import jax
import jax.numpy as jnp
from jax import lax
from jax.experimental import pallas as pl
from jax.experimental.pallas import tpu as pltpu
from jax.experimental.pallas import tpu_sc as plsc

N = 50000
E = 800000
D_IN = 39
H = 128
ED = 2

BN = 2000
CH = 32
NCH = H // CH
PPC = NCH // 2
B = 250
SUB = 125
NSUB = B // SUB
NBLK = E // B
NSC = 16
BLK_PER_TILE = NBLK // NSC
RPT = 3128
RPT_LAST = N - 15 * RPT


def _input_tables_call(xp, Wp, b_in2, Wcat, bcat2):
    BT = 1000
    grid = (N // BT,)
    outs = ([jax.ShapeDtypeStruct((N, H), jnp.float32)] +
            [jax.ShapeDtypeStruct((N, CH), jnp.float32) for _ in range(4 * NCH)])

    def body(x_ref, wp_ref, b_ref, wc_ref, bc_ref, h_ref, *tab_refs):
        hb = jnp.maximum(
            jnp.dot(x_ref[...], wp_ref[...], preferred_element_type=jnp.float32)
            + b_ref[...], 0.0)
        h_ref[...] = hb
        full = jnp.dot(hb, wc_ref[...], preferred_element_type=jnp.float32) + bc_ref[...]
        for t in range(4 * NCH):
            tab_refs[t][...] = full[:, t * CH:(t + 1) * CH]

    return pl.pallas_call(
        body,
        grid=grid,
        in_specs=[
            pl.BlockSpec((BT, H), lambda i: (i, 0)),
            pl.BlockSpec((H, H), lambda i: (0, 0)),
            pl.BlockSpec((1, H), lambda i: (0, 0)),
            pl.BlockSpec((H, 4 * NCH * CH), lambda i: (0, 0)),
            pl.BlockSpec((1, 4 * NCH * CH), lambda i: (0, 0)),
        ],
        out_specs=([pl.BlockSpec((BT, H), lambda i: (i, 0))] +
                   [pl.BlockSpec((BT, CH), lambda i: (i, 0))] * (4 * NCH)),
        out_shape=outs,
    )(xp, Wp, b_in2, Wcat, bcat2)


def _tables_call(h, Wcat, bcat2):
    BT = 1000
    grid = (N // BT,)
    outs = [jax.ShapeDtypeStruct((N, CH), jnp.float32) for _ in range(4 * NCH)]

    def body(h_ref, wc_ref, bc_ref, *tab_refs):
        full = jnp.dot(h_ref[...], wc_ref[...],
                       preferred_element_type=jnp.float32) + bc_ref[...]
        for t in range(4 * NCH):
            tab_refs[t][...] = full[:, t * CH:(t + 1) * CH]

    return pl.pallas_call(
        body,
        grid=grid,
        in_specs=[
            pl.BlockSpec((BT, H), lambda i: (i, 0)),
            pl.BlockSpec((H, 4 * NCH * CH), lambda i: (0, 0)),
            pl.BlockSpec((1, 4 * NCH * CH), lambda i: (0, 0)),
        ],
        out_specs=[pl.BlockSpec((BT, CH), lambda i: (i, 0))] * (4 * NCH),
        out_shape=outs,
    )(h, Wcat, bcat2)


def _combine_call(h, sa, se, dega2, dege2, aW2, ab2, eW2, eb2, g2, bl2, cW, cb2):
    grid = (N // BN,)

    def body(h_ref, sa_ref, se_ref, da_ref, de_ref, aw_ref, ab_ref,
             ew_ref, eb_ref, g_ref, bl_ref, cw_ref, cb_ref, o_ref):
        sa_full = jnp.concatenate([sa_ref[c] for c in range(NCH)], axis=-1)
        se_full = jnp.concatenate([se_ref[c] for c in range(NCH)], axis=-1)
        h1 = (h_ref[...]
              + jnp.dot(sa_full, aw_ref[...], preferred_element_type=jnp.float32)
              + da_ref[0, 0][:, None] * ab_ref[...]
              + jnp.dot(se_full, ew_ref[...], preferred_element_type=jnp.float32)
              + de_ref[0, 0][:, None] * eb_ref[...])
        mu = jnp.mean(h1, axis=-1, keepdims=True)
        var = jnp.mean((h1 - mu) ** 2, axis=-1, keepdims=True)
        t = (h1 - mu) / jnp.sqrt(var + 1e-5) * g_ref[...] + bl_ref[...]
        o_ref[...] = t + jnp.maximum(
            jnp.dot(t, cw_ref[...], preferred_element_type=jnp.float32)
            + cb_ref[...], 0.0)

    return pl.pallas_call(
        body,
        grid=grid,
        in_specs=[
            pl.BlockSpec((BN, H), lambda i: (i, 0)),
            pl.BlockSpec((NCH, BN, CH), lambda i: (0, i, 0)),
            pl.BlockSpec((NCH, BN, CH), lambda i: (0, i, 0)),
            pl.BlockSpec((1, 1, BN), lambda i: (i, 0, 0)),
            pl.BlockSpec((1, 1, BN), lambda i: (i, 0, 0)),
            pl.BlockSpec((H, H), lambda i: (0, 0)),
            pl.BlockSpec((1, H), lambda i: (0, 0)),
            pl.BlockSpec((H, H), lambda i: (0, 0)),
            pl.BlockSpec((1, H), lambda i: (0, 0)),
            pl.BlockSpec((1, H), lambda i: (0, 0)),
            pl.BlockSpec((1, H), lambda i: (0, 0)),
            pl.BlockSpec((H, H), lambda i: (0, 0)),
            pl.BlockSpec((1, H), lambda i: (0, 0)),
        ],
        out_specs=pl.BlockSpec((BN, H), lambda i: (i, 0)),
        out_shape=jax.ShapeDtypeStruct((N, H), jnp.float32),
    )(h, sa, se, dega2, dege2, aW2, ab2, eW2, eb2, g2, bl2, cW, cb2)


def _make_conv(want_deg):
    mesh = plsc.VectorSubcoreMesh(core_axis_name="c", subcore_axis_name="s",
                                  num_cores=2, num_subcores=NSC)
    out_type = [jax.ShapeDtypeStruct((NCH, N, CH), jnp.float32)]
    if want_deg:
        out_type.append(jax.ShapeDtypeStruct((N,), jnp.float32))
    scratch = [
        pltpu.VMEM_SHARED((N, CH), jnp.float32),
        pltpu.VMEM_SHARED((N,), jnp.float32),
        pltpu.VMEM((3, 2, B), jnp.int32),
        pltpu.VMEM((3, 2, B), jnp.float32),
        pltpu.VMEM((ED, CH), jnp.float32),
        pltpu.VMEM((2, B, CH), jnp.float32),
        pltpu.VMEM((B,), jnp.float32),
        pltpu.SemaphoreType.DMA,
        pltpu.SemaphoreType.DMA,
        pltpu.SemaphoreType.DMA,
    ]

    def body(*args):
        pts = args[0:NCH]
        qts = args[NCH:2 * NCH]
        (ids, eab, wattr, z2d, z1d, ones_h, s_out) = args[2 * NCH:2 * NCH + 7]
        rest = args[2 * NCH + 7:]
        if want_deg:
            deg_out = rest[0]
            rest = rest[1:]
        (acc, deg_acc, idx_b, ea_b, w_v, tb, ones_v,
         sem_f, sem_g, sem_s) = rest
        cid = lax.axis_index("c")
        sid = lax.axis_index("s")
        row0 = sid * RPT
        pltpu.sync_copy(ones_h, ones_v)

        def do_pass(chunk, with_deg):
            pt = pts[chunk]
            qt = qts[chunk]

            @pl.when(sid < NSC - 1)
            def _():
                pltpu.sync_copy(z2d, acc.at[pl.ds(row0, RPT)])

            @pl.when(sid == NSC - 1)
            def _():
                pltpu.sync_copy(z2d.at[pl.ds(0, RPT_LAST)],
                                acc.at[pl.ds(row0, RPT_LAST)])

            if with_deg:
                @pl.when(sid == 0)
                def _():
                    pltpu.sync_copy(z1d, deg_acc)
            pltpu.sync_copy(wattr.at[chunk], w_v)
            plsc.subcore_barrier()
            wv0 = [w_v[0, pl.ds(16 * h, 16)] for h in range(CH // 16)]
            wv1 = [w_v[1, pl.ds(16 * h, 16)] for h in range(CH // 16)]
            blk0 = sid * BLK_PER_TILE
            NB = BLK_PER_TILE

            def fetch_start(s3, j):
                pltpu.async_copy(ids.at[j], idx_b.at[s3], sem_f)
                pltpu.async_copy(eab.at[j], ea_b.at[s3], sem_f)

            def fetch_wait(s3):
                pltpu.make_async_copy(ids.at[0], idx_b.at[s3], sem_f).wait()
                pltpu.make_async_copy(eab.at[0], ea_b.at[s3], sem_f).wait()

            def gather_start(s2, s3):
                pltpu.async_copy(pt.at[idx_b.at[s3, 0]], tb.at[s2], sem_g,
                                 add=True)
                pltpu.async_copy(qt.at[idx_b.at[s3, 1]], tb.at[s2], sem_g,
                                 add=True)

            def gather_wait(s2, s3):
                pltpu.make_async_copy(pt.at[idx_b.at[s3, 0]], tb.at[s2],
                                      sem_g).wait()
                pltpu.make_async_copy(qt.at[idx_b.at[s3, 1]], tb.at[s2],
                                      sem_g).wait()

            def scat_start(s2, s3):
                pltpu.async_copy(tb.at[s2], acc.at[idx_b.at[s3, 0]], sem_s,
                                 add=True)
                if with_deg:
                    pltpu.async_copy(ones_v, deg_acc.at[idx_b.at[s3, 0]],
                                     sem_s, add=True)

            def scat_wait(s2, s3):
                pltpu.make_async_copy(tb.at[s2], acc.at[idx_b.at[s3, 0]],
                                      sem_s).wait()
                if with_deg:
                    pltpu.make_async_copy(ones_v, deg_acc.at[idx_b.at[s3, 0]],
                                          sem_s).wait()

            def attr_fill(s2, s3):
                @pl.loop(0, (B + 15) // 16, unroll=2)
                def _grp(g):
                    e0 = jnp.minimum(g * 16, B - 16)
                    a0v = ea_b[s3, 0, pl.ds(e0, 16)]
                    a1v = ea_b[s3, 1, pl.ds(e0, 16)]
                    for i in range(16):
                        for h in range(CH // 16):
                            tb[s2, e0 + i, pl.ds(16 * h, 16)] = (
                                a0v[i] * wv0[h] + a1v[i] * wv1[h])

            def relu_inplace(s2):
                @pl.loop(0, B, unroll=8)
                def _r(i):
                    for h in range(CH // 16):
                        v = tb[s2, i, pl.ds(16 * h, 16)]
                        tb[s2, i, pl.ds(16 * h, 16)] = jnp.maximum(v, 0.0)

            fetch_start(0, blk0)
            fetch_wait(0)
            attr_fill(0, 0)
            gather_start(0, 0)

            @pl.when(NB > 1)
            def _():
                fetch_start(1, blk0 + 1)

            @pl.loop(0, NB)
            def _blk(jj):
                j = blk0 + jj
                s2 = jax.lax.rem(jj, 2)
                s3 = jax.lax.rem(jj, 3)
                s2n = jax.lax.rem(jj + 1, 2)
                s3n = jax.lax.rem(jj + 1, 3)
                s3nn = jax.lax.rem(jj + 2, 3)

                @pl.when(jj < NB - 1)
                def _():
                    fetch_wait(s3n)

                    @pl.when(jj >= 1)
                    def _():
                        scat_wait(s2n, s3n)
                    attr_fill(s2n, s3n)
                    gather_start(s2n, s3n)

                @pl.when(jj < NB - 2)
                def _():
                    fetch_start(s3nn, j + 2)

                gather_wait(s2, s3)
                relu_inplace(s2)
                scat_start(s2, s3)

            @pl.when(NB > 1)
            def _():
                scat_wait(jax.lax.rem(NB - 2, 2), jax.lax.rem(NB - 2, 3))
            scat_wait(jax.lax.rem(NB - 1, 2), jax.lax.rem(NB - 1, 3))

            plsc.subcore_barrier()

            @pl.when(sid < NSC - 1)
            def _():
                pltpu.sync_copy(acc.at[pl.ds(row0, RPT)],
                                s_out.at[chunk].at[pl.ds(row0, RPT)])

            @pl.when(sid == NSC - 1)
            def _():
                pltpu.sync_copy(acc.at[pl.ds(row0, RPT_LAST)],
                                s_out.at[chunk].at[pl.ds(row0, RPT_LAST)])
            if with_deg:
                @pl.when(sid == 0)
                def _():
                    pltpu.sync_copy(deg_acc, deg_out)

        @pl.when(cid == 0)
        def _():
            for p in range(PPC):
                do_pass(p, want_deg and p == 0)

        @pl.when(cid == 1)
        def _():
            for p in range(PPC, NCH):
                do_pass(p, False)

    return pl.kernel(body, out_type=out_type, mesh=mesh,
                     scratch_types=scratch,
                     compiler_params=pltpu.CompilerParams(
                         use_tc_tiling_on_sc=False))


_conv_cache = {}


def _get_conv(want_deg):
    if want_deg not in _conv_cache:
        _conv_cache[want_deg] = _make_conv(want_deg)
    return _conv_cache[want_deg]


def kernel(x, ally_edge_index, ally_edge_attr, enc_edge_index, enc_edge_attr,
           W_in, b_in, ally_W1_0, ally_b1_0, ally_W2_0, ally_b2_0,
           enc_W1_0, enc_b1_0, enc_W2_0, enc_b2_0, ln_g_0, ln_b_0,
           comb_W_0, comb_b_0, ally_W1_1, ally_b1_1, ally_W2_1, ally_b2_1,
           enc_W1_1, enc_b1_1, enc_W2_1, enc_b2_1, ln_g_1, ln_b_1,
           comb_W_1, comb_b_1):
    f32 = jnp.float32
    conv_deg = _get_conv(True)
    conv_nodeg = _get_conv(False)

    xp = jnp.pad(x, ((0, 0), (0, H - D_IN)))
    Wp = jnp.pad(W_in, ((0, H - D_IN), (0, 0)))

    def edge_prep(ei, ea):
        dst3 = ei[1].reshape(NBLK, 1, B)
        src3 = ei[0].reshape(NBLK, 1, B)
        ids = jnp.concatenate([dst3, src3], axis=1)
        eab = jnp.stack([ea[:, 0].reshape(NBLK, B),
                         ea[:, 1].reshape(NBLK, B)], axis=1)
        return ids, eab

    a_ids, a_eab = edge_prep(ally_edge_index, ally_edge_attr)
    e_ids, e_eab = edge_prep(enc_edge_index, enc_edge_attr)

    def wcat_prep(aW1, ab1, eW1, eb1):
        Wcat = jnp.concatenate(
            [aW1[:H], aW1[H:2 * H], eW1[:H], eW1[H:2 * H]], axis=1)
        bcat = jnp.concatenate(
            [ab1, jnp.zeros((H,), f32), eb1, jnp.zeros((H,), f32)])
        wattr_a = aW1[2 * H:].reshape(ED, NCH, CH).transpose(1, 0, 2)
        wattr_e = eW1[2 * H:].reshape(ED, NCH, CH).transpose(1, 0, 2)
        return Wcat, bcat.reshape(1, -1), wattr_a, wattr_e

    Wcat0, bcat0, wattr_a0, wattr_e0 = wcat_prep(ally_W1_0, ally_b1_0,
                                                 enc_W1_0, enc_b1_0)
    Wcat1, bcat1, wattr_a1, wattr_e1 = wcat_prep(ally_W1_1, ally_b1_1,
                                                 enc_W1_1, enc_b1_1)

    z2d = jnp.zeros((RPT, CH), f32)
    z1d = jnp.zeros((N,), f32)
    ones_h = jnp.ones((B,), f32)

    def r2(v):
        return v.reshape(1, -1)

    outs = _input_tables_call(xp, Wp, r2(b_in), Wcat0, bcat0)
    h = outs[0]
    tabs = outs[1:]
    ap, aq = tabs[0:NCH], tabs[NCH:2 * NCH]
    ep, eq = tabs[2 * NCH:3 * NCH], tabs[3 * NCH:4 * NCH]

    sa, dega = conv_deg(*ap, *aq, a_ids, a_eab,
                        wattr_a0, z2d, z1d, ones_h)
    se, dege = conv_deg(*ep, *eq, e_ids, e_eab,
                        wattr_e0, z2d, z1d, ones_h)

    dega2 = dega.reshape(N // BN, 1, BN)
    dege2 = dege.reshape(N // BN, 1, BN)

    h = _combine_call(h, sa, se, dega2, dege2,
                      ally_W2_0, r2(ally_b2_0), enc_W2_0, r2(enc_b2_0),
                      r2(ln_g_0), r2(ln_b_0), comb_W_0, r2(comb_b_0))

    tabs = _tables_call(h, Wcat1, bcat1)
    ap, aq = tabs[0:NCH], tabs[NCH:2 * NCH]
    ep, eq = tabs[2 * NCH:3 * NCH], tabs[3 * NCH:4 * NCH]

    (sa,) = conv_nodeg(*ap, *aq, a_ids, a_eab,
                       wattr_a1, z2d, z1d, ones_h)
    (se,) = conv_nodeg(*ep, *eq, e_ids, e_eab,
                       wattr_e1, z2d, z1d, ones_h)

    h = _combine_call(h, sa, se, dega2, dege2,
                      ally_W2_1, r2(ally_b2_1), enc_W2_1, r2(enc_b2_1),
                      r2(ln_g_1), r2(ln_b_1), comb_W_1, r2(comb_b_1))
    return h
